# baseline (speedup 1.0000x reference)
import functools
import math

import jax
import jax.numpy as jnp
from jax import lax
from jax.experimental import pallas as pl
from jax.experimental.pallas import tpu as pltpu
from jax.experimental.pallas import tpu_sc as plsc

N = 10000
E = 160000
D = 128
H = 1024

NP = 10240
F = 128
EP = 163840
EPT = EP // 32
NBATCH = EPT // 128
NSTRIPE = NP // 16

_f32 = jnp.float32
_i32 = jnp.int32


def _make_edge_agg(nch):
  mesh = plsc.VectorSubcoreMesh(core_axis_name="c", subcore_axis_name="s")

  def body(x3, srcef, dst2, val2, counts, aggp, cntp,
           aggsp, cntsp, idxf, idxb, dstst, valst, gbuf, cbuf, ctile, sem):
    cid = lax.axis_index("c")
    sid = lax.axis_index("s")
    tid = cid * 16 + sid
    ebase = pl.multiple_of(cid * (EP // 2) + sid * EPT, EPT)
    rbase = pl.multiple_of(ebase // 128, NBATCH)
    r0 = pl.multiple_of(sid * NSTRIPE, NSTRIPE)

    pltpu.sync_copy(srcef.at[pl.ds(ebase, EPT)], idxf)
    pltpu.sync_copy(dst2.at[pl.ds(rbase, NBATCH)], dstst)
    pltpu.sync_copy(val2.at[pl.ds(rbase, NBATCH)], valst)
    pltpu.sync_copy(counts.at[tid], ctile)
    nb = (ctile[...][0] + 127) // 128

    for ch in range(nch):
      def zfill(t, carry):
        gbuf[t // 8, pl.ds((t % 8) * 16, 16)] = jnp.zeros((16,), _f32)
        return carry
      lax.fori_loop(0, 128 * 8, zfill, 0)
      for m in range(NSTRIPE // 128):
        pltpu.sync_copy(gbuf, aggsp.at[pl.ds(r0 + m * 128, 128)])
      if ch == 0:
        for m in range(NSTRIPE // 128):
          pltpu.sync_copy(gbuf.at[0], cntsp.at[pl.ds(r0 + m * 128, 128)])
      plsc.subcore_barrier()

      coff = ch * NP

      def batch(j, carry):
        if nch > 1:
          def afill(t, carry2):
            idxb[pl.ds(t * 16, 16)] = idxf[pl.ds(j * 128 + t * 16, 16)] + coff
            return carry2
          lax.fori_loop(0, 8, afill, 0)
          idxsrc = idxb
        else:
          idxsrc = idxf.at[pl.ds(j * 128, 128)]
        pltpu.async_copy(x3.at[idxsrc], gbuf, sem).wait()
        pltpu.sync_copy(gbuf, aggsp.at[dstst.at[j]], add=True)
        return carry
      lax.fori_loop(0, nb, batch, 0)

      if ch == 0:
        def cbatch(j, carry):
          pltpu.sync_copy(valst.at[j], cntsp.at[dstst.at[j]], add=True)
          return carry
        lax.fori_loop(0, nb, cbatch, 0)

      plsc.subcore_barrier()

      for m in range(NSTRIPE // 128):
        pltpu.sync_copy(aggsp.at[pl.ds(r0 + m * 128, 128)], gbuf)
        pltpu.sync_copy(
            gbuf, aggp.at[cid, pl.ds(r0 + m * 128, 128), pl.ds(ch * F, F)])
      if ch == 0:
        pltpu.sync_copy(cntsp.at[pl.ds(r0, NSTRIPE)], cbuf)
        pltpu.sync_copy(cbuf, cntp.at[cid, pl.ds(r0, NSTRIPE)])

  return pl.kernel(
      body,
      out_type=[
          jax.ShapeDtypeStruct((2, NP, nch * F), _f32),
          jax.ShapeDtypeStruct((2, NP), _f32),
      ],
      mesh=mesh,
      scratch_types=[
          pltpu.VMEM_SHARED((NP, F), _f32),
          pltpu.VMEM_SHARED((NP,), _f32),
          pltpu.VMEM((EPT,), _i32),
          pltpu.VMEM((128,), _i32),
          pltpu.VMEM((NBATCH, 128), _i32),
          pltpu.VMEM((NBATCH, 128), _f32),
          pltpu.VMEM((128, F), _f32),
          pltpu.VMEM((NSTRIPE,), _f32),
          pltpu.VMEM((16,), _i32),
          pltpu.SemaphoreType.DMA,
      ],
      name=f"edge_agg_{nch}",
  )


def _make_revalidate():
  mesh = plsc.VectorSubcoreMesh(core_axis_name="c", subcore_axis_name="s")

  def body(keep, srcp, dstp, valp, srcc_o, dstc_o, valc_o, valn_o, cnts_o,
           keepst, sst, dstt, vst_, sout, dout, vout, vfull, cb16):
    cid = lax.axis_index("c")
    sid = lax.axis_index("s")
    tid = cid * 16 + sid
    ebase = pl.multiple_of(cid * (EP // 2) + sid * EPT, EPT)

    pltpu.sync_copy(keep, keepst)
    pltpu.sync_copy(srcp.at[pl.ds(ebase, EPT)], sst)
    pltpu.sync_copy(dstp.at[pl.ds(ebase, EPT)], dstt)
    pltpu.sync_copy(valp.at[pl.ds(ebase, EPT)], vst_)

    iota16 = lax.iota(_i32, 16)
    zeros16i = jnp.zeros((16,), _i32)
    ones16 = jnp.ones((16,), _f32)

    def pre(t, c):
      sout[pl.ds(t * 16, 16)] = N + ((iota16 + t) % 32)
      dout[pl.ds(t * 16, 16)] = zeros16i + (NP - 1)
      vout[pl.ds(t * 16, 16)] = jnp.zeros((16,), _f32)
      return c
    lax.fori_loop(0, EPT // 16, pre, 0)

    def step(t, cnt):
      s16 = sst[pl.ds(t * 16, 16)]
      d16 = dstt[pl.ds(t * 16, 16)]
      v16 = vst_[pl.ds(t * 16, 16)]
      ks = plsc.load_gather(keepst, [s16])
      kd = plsc.load_gather(keepst, [d16])
      v = v16 * ks * kd
      vfull[pl.ds(t * 16, 16)] = v
      m = v > 0.0
      vi = jnp.where(m, 1, 0)
      pos = plsc.cumsum(vi) + (cnt - 1)
      plsc.store_scatter(sout, [pos], s16, mask=m)
      plsc.store_scatter(dout, [pos], d16, mask=m)
      plsc.store_scatter(vout, [pos], ones16, mask=m)
      return cnt + jnp.sum(vi)
    cnt = lax.fori_loop(0, EPT // 16, step, 0)

    cb16[...] = zeros16i + cnt
    pltpu.sync_copy(sout, srcc_o.at[pl.ds(ebase, EPT)])
    pltpu.sync_copy(dout, dstc_o.at[pl.ds(ebase, EPT)])
    pltpu.sync_copy(vout, valc_o.at[pl.ds(ebase, EPT)])
    pltpu.sync_copy(vfull, valn_o.at[pl.ds(ebase, EPT)])
    pltpu.sync_copy(cb16, cnts_o.at[tid])

  return pl.kernel(
      body,
      out_type=[
          jax.ShapeDtypeStruct((EP,), _i32),
          jax.ShapeDtypeStruct((EP,), _i32),
          jax.ShapeDtypeStruct((EP,), _f32),
          jax.ShapeDtypeStruct((EP,), _f32),
          jax.ShapeDtypeStruct((32, 16), _i32),
      ],
      mesh=mesh,
      scratch_types=[
          pltpu.VMEM((NP,), _f32),
          pltpu.VMEM((EPT,), _i32),
          pltpu.VMEM((EPT,), _i32),
          pltpu.VMEM((EPT,), _f32),
          pltpu.VMEM((EPT,), _i32),
          pltpu.VMEM((EPT,), _i32),
          pltpu.VMEM((EPT,), _f32),
          pltpu.VMEM((EPT,), _f32),
          pltpu.VMEM((16,), _i32),
      ],
      compiler_params=pltpu.CompilerParams(needs_layout_passes=False),
      name="revalidate",
  )


def _make_xr(din):
  bm = 256

  def body(x, wr, b, xr_ref):
    xr_ref[...] = (jnp.dot(x[...], wr[...], preferred_element_type=_f32)
                   + b[...])

  return pl.pallas_call(
      body,
      grid=(NP // bm,),
      in_specs=[
          pl.BlockSpec((bm, din), lambda i: (i, 0)),
          pl.BlockSpec((din, H), lambda i: (0, 0)),
          pl.BlockSpec((1, H), lambda i: (0, 0)),
      ],
      out_specs=pl.BlockSpec((bm, H), lambda i: (i, 0)),
      out_shape=jax.ShapeDtypeStruct((NP, H), _f32),
      name=f"xr_{din}",
  )


def _make_sage(din):
  bm = 256

  def body(aggp, cntp, xr, wl, p, h_ref, sraw_ref):
    agg = aggp[0] + aggp[1]
    c = cntp[0] + cntp[1]
    mean = jnp.where(c > 0.0, agg / jnp.maximum(c, 1.0), 0.0)
    hm = jnp.dot(mean, wl[...], preferred_element_type=_f32) + xr[...]
    h = jnp.maximum(hm, 0.0)
    h_ref[...] = h
    sraw_ref[...] = jnp.dot(h, p[...], preferred_element_type=_f32)

  return pl.pallas_call(
      body,
      grid=(NP // bm,),
      in_specs=[
          pl.BlockSpec((2, bm, din), lambda i: (0, i, 0)),
          pl.BlockSpec((2, bm, 1), lambda i: (0, i, 0)),
          pl.BlockSpec((bm, H), lambda i: (i, 0)),
          pl.BlockSpec((din, H), lambda i: (0, 0)),
          pl.BlockSpec((H, 1), lambda i: (0, 0)),
      ],
      out_specs=[
          pl.BlockSpec((bm, H), lambda i: (i, 0)),
          pl.BlockSpec((bm, 1), lambda i: (i, 0)),
      ],
      out_shape=[
          jax.ShapeDtypeStruct((NP, H), _f32),
          jax.ShapeDtypeStruct((NP, 1), _f32),
      ],
      name=f"sage_{din}",
  )


def _make_topk(k):
  def body(sraw, keep, p, keepn_ref, scale_ref):
    s = sraw[...]
    kp = keep[...]
    bits = lax.bitcast_convert_type(s, jnp.uint32)
    top = jnp.uint32(0x80000000)
    sortable = jnp.where((bits & top) != 0, ~bits, bits | top)
    hi = jnp.where(kp > 0.0, sortable, jnp.uint32(0))
    ridx = (lax.broadcasted_iota(_i32, (80, 128), 0) * 128
            + lax.broadcasted_iota(_i32, (80, 128), 1))
    lo = (NP - ridx).astype(jnp.uint32)

    def hstep(t, pref):
      cand = pref | (jnp.uint32(1) << (31 - t).astype(jnp.uint32))
      cnt = jnp.sum((hi >= cand).astype(_i32))
      return jnp.where(cnt >= k, cand, pref)
    hstar = lax.fori_loop(0, 32, hstep, jnp.uint32(0))

    ngt = jnp.sum((hi > hstar).astype(_i32))
    r = k - ngt
    tie = hi == hstar

    def lstep(t, pref):
      cand = pref | (jnp.uint32(1) << (13 - t).astype(jnp.uint32))
      cnt = jnp.sum((tie & (lo >= cand)).astype(_i32))
      return jnp.where(cnt >= r, cand, pref)
    lstar = lax.fori_loop(0, 14, lstep, jnp.uint32(0))

    keepn = ((hi > hstar) | (tie & (lo >= lstar))).astype(_f32)
    keepn_ref[...] = keepn
    pn = jnp.sqrt(jnp.sum(p[...] * p[...]))
    scale_ref[...] = jnp.tanh(s / (pn + 1e-16)) * keepn

  return pl.pallas_call(
      body,
      out_shape=[
          jax.ShapeDtypeStruct((80, 128), _f32),
          jax.ShapeDtypeStruct((80, 128), _f32),
      ],
      name=f"topk_{k}",
  )


def _make_pool(kn):
  bm = 256
  nrow = NP // bm

  def body(h, scale, keep, x3_ref, xrow_ref, ro_ref):
    i = pl.program_id(1)
    xn = h[...] * scale[...]
    x3_ref[...] = xn
    xrow_ref[...] = xn
    masked = jnp.where(keep[...] > 0.0, xn, -3.4e38)
    cmax = jnp.max(masked, axis=0, keepdims=True)
    csum = jnp.sum(xn, axis=0, keepdims=True)

    @pl.when(i == 0)
    def _():
      ro_ref[...] = jnp.concatenate([cmax, csum], axis=0)[None]

    @pl.when(i > 0)
    def _():
      cur = ro_ref[...]
      mx = jnp.maximum(cur[0, 0:1], cmax)
      sm = cur[0, 1:2] + csum
      ro_ref[...] = jnp.concatenate([mx, sm], axis=0)[None]

    @pl.when(i == nrow - 1)
    def _():
      cur = ro_ref[...]
      ro_ref[...] = jnp.concatenate(
          [cur[0, 0:1], cur[0, 1:2] * (1.0 / kn)], axis=0)[None]

  return pl.pallas_call(
      body,
      grid=(8, nrow),
      in_specs=[
          pl.BlockSpec((bm, F), lambda c, i: (i, c)),
          pl.BlockSpec((bm, 1), lambda c, i: (i, 0)),
          pl.BlockSpec((bm, 1), lambda c, i: (i, 0)),
      ],
      out_specs=[
          pl.BlockSpec((bm, F), lambda c, i: (c * nrow + i, 0)),
          pl.BlockSpec((bm, F), lambda c, i: (i, c)),
          pl.BlockSpec((1, 2, F), lambda c, i: (c, 0, 0)),
      ],
      out_shape=[
          jax.ShapeDtypeStruct((8 * NP, F), _f32),
          jax.ShapeDtypeStruct((NP, H), _f32),
          jax.ShapeDtypeStruct((8, 2, F), _f32),
      ],
      name=f"pool_{kn}",
  )


_edge_agg_1 = _make_edge_agg(1)
_edge_agg_8 = _make_edge_agg(8)
_revalidate = _make_revalidate()
_xr_d = _make_xr(D)
_xr_h = _make_xr(H)
_sage_d = _make_sage(D)
_sage_h = _make_sage(H)
_topk = {k: _make_topk(k) for k in (8000, 6400, 5120)}
_pool = {k: _make_pool(k) for k in (8000, 6400, 5120)}


def kernel(x, edge_index, batch, w1_l, w1_r, b1, p1, w2_l, w2_r, b2, p2,
           w3_l, w3_r, b3, p3):
  del batch
  xp = jnp.zeros((NP, D), _f32).at[:N].set(x)
  src = edge_index[0].astype(_i32)
  dst = edge_index[1].astype(_i32)
  npad = EP - E
  pad_dum = N + (jnp.arange(npad, dtype=_i32) % 32)
  srcp = jnp.concatenate([src, pad_dum])
  dstp = jnp.concatenate([dst, jnp.full((npad,), NP - 1, _i32)])
  valid = jnp.concatenate([jnp.ones((E,), _f32), jnp.zeros((npad,), _f32)])
  dst2 = dstp.reshape(EP // 128, 128)
  keep = jnp.concatenate([jnp.ones((N,), _f32), jnp.zeros((NP - N,), _f32)])

  srcef = srcp
  dstcur = dstp
  valtile = valid
  counts = jnp.full((32, 16), EPT, _i32)
  x3 = xp
  xrow = xp
  result = jnp.zeros((1, 2 * H), _f32)

  layers = [
      (w1_l, w1_r, b1, p1, _sage_d, _xr_d, _edge_agg_1, 8000),
      (w2_l, w2_r, b2, p2, _sage_h, _xr_h, _edge_agg_8, 6400),
      (w3_l, w3_r, b3, p3, _sage_h, _xr_h, _edge_agg_8, 5120),
  ]
  for li, (wl, wr, b, p, sage, xrk, eagg, kn) in enumerate(layers):
    xr = xrk(xrow, wr, b.reshape(1, H))
    aggp, cntp = eagg(x3, srcef, dstcur.reshape(EP // 128, 128),
                      valtile.reshape(EP // 128, 128), counts)
    h, sraw = sage(aggp, cntp.reshape(2, NP, 1), xr, wl, p.reshape(H, 1))
    keepn2, scale2 = _topk[kn](sraw.reshape(80, 128), keep.reshape(80, 128),
                               p.reshape(8, 128))
    keepn = keepn2.reshape(NP)
    x3, xrow, ro = _pool[kn](h, scale2.reshape(NP, 1), keepn.reshape(NP, 1))
    result = result + jnp.concatenate(
        [ro[:, 0].reshape(1, H), ro[:, 1].reshape(1, H)], axis=1)
    if li < 2:
      srcef, dstcur, valtile, valid, counts = _revalidate(
          keepn, srcp, dstp, valid)
      keep = keepn
  return result


# speedup vs baseline: 1.0093x; 1.0093x over previous
import functools
import math

import jax
import jax.numpy as jnp
from jax import lax
from jax.experimental import pallas as pl
from jax.experimental.pallas import tpu as pltpu
from jax.experimental.pallas import tpu_sc as plsc

N = 10000
E = 160000
D = 128
H = 1024

NP = 10240
F = 128
EP = 163840
EPT = EP // 32
NBATCH = EPT // 128
NSTRIPE = NP // 16

_f32 = jnp.float32
_i32 = jnp.int32


def _make_edge_agg(nch):
  mesh = plsc.VectorSubcoreMesh(core_axis_name="c", subcore_axis_name="s")

  def body(x3, srcef, dst2, val2, counts, aggp, cntp,
           aggsp, cntsp, idxf, idxb, dstst, valst, gbuf, cbuf, ctile, sem):
    cid = lax.axis_index("c")
    sid = lax.axis_index("s")
    tid = cid * 16 + sid
    ebase = pl.multiple_of(cid * (EP // 2) + sid * EPT, EPT)
    rbase = pl.multiple_of(ebase // 128, NBATCH)
    r0 = pl.multiple_of(sid * NSTRIPE, NSTRIPE)

    pltpu.sync_copy(srcef.at[pl.ds(ebase, EPT)], idxf)
    pltpu.sync_copy(dst2.at[pl.ds(rbase, NBATCH)], dstst)
    pltpu.sync_copy(val2.at[pl.ds(rbase, NBATCH)], valst)
    pltpu.sync_copy(counts.at[tid], ctile)
    nb = (ctile[...][0] + 127) // 128

    for ch in range(nch):
      def zfill(t, carry):
        gbuf[t // 8, pl.ds((t % 8) * 16, 16)] = jnp.zeros((16,), _f32)
        return carry
      lax.fori_loop(0, 128 * 8, zfill, 0)
      for m in range(NSTRIPE // 128):
        pltpu.sync_copy(gbuf, aggsp.at[pl.ds(r0 + m * 128, 128)])
      if ch == 0:
        for m in range(NSTRIPE // 128):
          pltpu.sync_copy(gbuf.at[0], cntsp.at[pl.ds(r0 + m * 128, 128)])
      plsc.subcore_barrier()

      coff = ch * NP

      def batch(j, carry):
        if nch > 1:
          def afill(t, carry2):
            idxb[pl.ds(t * 16, 16)] = idxf[pl.ds(j * 128 + t * 16, 16)] + coff
            return carry2
          lax.fori_loop(0, 8, afill, 0)
          idxsrc = idxb
        else:
          idxsrc = idxf.at[pl.ds(j * 128, 128)]
        pltpu.async_copy(x3.at[idxsrc], gbuf, sem).wait()
        pltpu.sync_copy(gbuf, aggsp.at[dstst.at[j]], add=True)
        return carry
      lax.fori_loop(0, nb, batch, 0)

      if ch == 0:
        def cbatch(j, carry):
          pltpu.sync_copy(valst.at[j], cntsp.at[dstst.at[j]], add=True)
          return carry
        lax.fori_loop(0, nb, cbatch, 0)

      plsc.subcore_barrier()

      for m in range(NSTRIPE // 128):
        pltpu.sync_copy(aggsp.at[pl.ds(r0 + m * 128, 128)], gbuf)
        pltpu.sync_copy(
            gbuf, aggp.at[cid, pl.ds(r0 + m * 128, 128), pl.ds(ch * F, F)])
      if ch == 0:
        pltpu.sync_copy(cntsp.at[pl.ds(r0, NSTRIPE)], cbuf)
        pltpu.sync_copy(cbuf, cntp.at[cid, pl.ds(r0, NSTRIPE)])

  return pl.kernel(
      body,
      out_type=[
          jax.ShapeDtypeStruct((2, NP, nch * F), _f32),
          jax.ShapeDtypeStruct((2, NP), _f32),
      ],
      mesh=mesh,
      scratch_types=[
          pltpu.VMEM_SHARED((NP, F), _f32),
          pltpu.VMEM_SHARED((NP,), _f32),
          pltpu.VMEM((EPT,), _i32),
          pltpu.VMEM((128,), _i32),
          pltpu.VMEM((NBATCH, 128), _i32),
          pltpu.VMEM((NBATCH, 128), _f32),
          pltpu.VMEM((128, F), _f32),
          pltpu.VMEM((NSTRIPE,), _f32),
          pltpu.VMEM((16,), _i32),
          pltpu.SemaphoreType.DMA,
      ],
      name=f"edge_agg_{nch}",
  )


def _make_revalidate():
  mesh = plsc.VectorSubcoreMesh(core_axis_name="c", subcore_axis_name="s")

  def body(keep, srcp, dstp, valp, srcc_o, dstc_o, valc_o, valn_o, cnts_o,
           keepst, sst, dstt, vst_, sout, dout, vout, vfull, cb16):
    cid = lax.axis_index("c")
    sid = lax.axis_index("s")
    tid = cid * 16 + sid
    ebase = pl.multiple_of(cid * (EP // 2) + sid * EPT, EPT)

    pltpu.sync_copy(keep, keepst)
    pltpu.sync_copy(srcp.at[pl.ds(ebase, EPT)], sst)
    pltpu.sync_copy(dstp.at[pl.ds(ebase, EPT)], dstt)
    pltpu.sync_copy(valp.at[pl.ds(ebase, EPT)], vst_)

    iota16 = lax.iota(_i32, 16)
    zeros16i = jnp.zeros((16,), _i32)
    ones16 = jnp.ones((16,), _f32)

    def pre(t, c):
      sout[pl.ds(t * 16, 16)] = N + ((iota16 + t) % 32)
      dout[pl.ds(t * 16, 16)] = zeros16i + (NP - 1)
      vout[pl.ds(t * 16, 16)] = jnp.zeros((16,), _f32)
      return c
    lax.fori_loop(0, EPT // 16, pre, 0)

    def step(t, cnt):
      s16 = sst[pl.ds(t * 16, 16)]
      d16 = dstt[pl.ds(t * 16, 16)]
      v16 = vst_[pl.ds(t * 16, 16)]
      ks = plsc.load_gather(keepst, [s16])
      kd = plsc.load_gather(keepst, [d16])
      v = v16 * ks * kd
      vfull[pl.ds(t * 16, 16)] = v
      m = v > 0.0
      vi = jnp.where(m, 1, 0)
      pos = plsc.cumsum(vi) + (cnt - 1)
      plsc.store_scatter(sout, [pos], s16, mask=m)
      plsc.store_scatter(dout, [pos], d16, mask=m)
      plsc.store_scatter(vout, [pos], ones16, mask=m)
      return cnt + jnp.sum(vi)
    cnt = lax.fori_loop(0, EPT // 16, step, 0)

    cb16[...] = zeros16i + cnt
    pltpu.sync_copy(sout, srcc_o.at[pl.ds(ebase, EPT)])
    pltpu.sync_copy(dout, dstc_o.at[pl.ds(ebase, EPT)])
    pltpu.sync_copy(vout, valc_o.at[pl.ds(ebase, EPT)])
    pltpu.sync_copy(vfull, valn_o.at[pl.ds(ebase, EPT)])
    pltpu.sync_copy(cb16, cnts_o.at[tid])

  return pl.kernel(
      body,
      out_type=[
          jax.ShapeDtypeStruct((EP,), _i32),
          jax.ShapeDtypeStruct((EP,), _i32),
          jax.ShapeDtypeStruct((EP,), _f32),
          jax.ShapeDtypeStruct((EP,), _f32),
          jax.ShapeDtypeStruct((32, 16), _i32),
      ],
      mesh=mesh,
      scratch_types=[
          pltpu.VMEM((NP,), _f32),
          pltpu.VMEM((EPT,), _i32),
          pltpu.VMEM((EPT,), _i32),
          pltpu.VMEM((EPT,), _f32),
          pltpu.VMEM((EPT,), _i32),
          pltpu.VMEM((EPT,), _i32),
          pltpu.VMEM((EPT,), _f32),
          pltpu.VMEM((EPT,), _f32),
          pltpu.VMEM((16,), _i32),
      ],
      compiler_params=pltpu.CompilerParams(needs_layout_passes=False),
      name="revalidate",
  )


def _make_xr(din):
  bm = 256

  def body(x, wr, b, xr_ref):
    xr_ref[...] = (jnp.dot(x[...], wr[...], preferred_element_type=_f32)
                   + b[...])

  return pl.pallas_call(
      body,
      grid=(NP // bm,),
      in_specs=[
          pl.BlockSpec((bm, din), lambda i: (i, 0)),
          pl.BlockSpec((din, H), lambda i: (0, 0)),
          pl.BlockSpec((1, H), lambda i: (0, 0)),
      ],
      out_specs=pl.BlockSpec((bm, H), lambda i: (i, 0)),
      out_shape=jax.ShapeDtypeStruct((NP, H), _f32),
      name=f"xr_{din}",
  )


def _make_xr_cm():
  bm = 256
  nrow = NP // bm

  def body(x3, wr, b, xr_ref):
    c = pl.program_id(1)

    @pl.when(c == 0)
    def _():
      xr_ref[...] = jnp.zeros((bm, H), _f32) + b[...]

    xr_ref[...] += jnp.dot(x3[...], wr[...], preferred_element_type=_f32)

  return pl.pallas_call(
      body,
      grid=(nrow, 8),
      in_specs=[
          pl.BlockSpec((bm, F), lambda i, c: (c * nrow + i, 0)),
          pl.BlockSpec((F, H), lambda i, c: (c, 0)),
          pl.BlockSpec((1, H), lambda i, c: (0, 0)),
      ],
      out_specs=pl.BlockSpec((bm, H), lambda i, c: (i, 0)),
      out_shape=jax.ShapeDtypeStruct((NP, H), _f32),
      name="xr_cm",
  )


def _make_sage(din):
  bm = 256

  def body(aggp, cntp, xr, wl, p, h_ref, sraw_ref):
    agg = aggp[0] + aggp[1]
    c = cntp[0] + cntp[1]
    mean = jnp.where(c > 0.0, agg / jnp.maximum(c, 1.0), 0.0)
    hm = jnp.dot(mean, wl[...], preferred_element_type=_f32) + xr[...]
    h = jnp.maximum(hm, 0.0)
    h_ref[...] = h
    sraw_ref[...] = jnp.dot(h, p[...], preferred_element_type=_f32)

  return pl.pallas_call(
      body,
      grid=(NP // bm,),
      in_specs=[
          pl.BlockSpec((2, bm, din), lambda i: (0, i, 0)),
          pl.BlockSpec((2, bm, 1), lambda i: (0, i, 0)),
          pl.BlockSpec((bm, H), lambda i: (i, 0)),
          pl.BlockSpec((din, H), lambda i: (0, 0)),
          pl.BlockSpec((H, 1), lambda i: (0, 0)),
      ],
      out_specs=[
          pl.BlockSpec((bm, H), lambda i: (i, 0)),
          pl.BlockSpec((bm, 1), lambda i: (i, 0)),
      ],
      out_shape=[
          jax.ShapeDtypeStruct((NP, H), _f32),
          jax.ShapeDtypeStruct((NP, 1), _f32),
      ],
      name=f"sage_{din}",
  )


def _make_topk(k):
  def body(sraw, keep, p, keepn_ref, scale_ref):
    s = sraw[...]
    kp = keep[...]
    bits = lax.bitcast_convert_type(s, jnp.uint32)
    top = jnp.uint32(0x80000000)
    sortable = jnp.where((bits & top) != 0, ~bits, bits | top)
    hi = jnp.where(kp > 0.0, sortable, jnp.uint32(0))
    ridx = (lax.broadcasted_iota(_i32, (80, 128), 0) * 128
            + lax.broadcasted_iota(_i32, (80, 128), 1))
    lo = (NP - ridx).astype(jnp.uint32)

    def hstep(t, pref):
      cand = pref | (jnp.uint32(1) << (31 - t).astype(jnp.uint32))
      cnt = jnp.sum((hi >= cand).astype(_i32))
      return jnp.where(cnt >= k, cand, pref)
    hstar = lax.fori_loop(0, 32, hstep, jnp.uint32(0))

    ngt = jnp.sum((hi > hstar).astype(_i32))
    r = k - ngt
    tie = hi == hstar

    def lstep(t, pref):
      cand = pref | (jnp.uint32(1) << (13 - t).astype(jnp.uint32))
      cnt = jnp.sum((tie & (lo >= cand)).astype(_i32))
      return jnp.where(cnt >= r, cand, pref)
    lstar = lax.fori_loop(0, 14, lstep, jnp.uint32(0))

    keepn = ((hi > hstar) | (tie & (lo >= lstar))).astype(_f32)
    keepn_ref[...] = keepn
    pn = jnp.sqrt(jnp.sum(p[...] * p[...]))
    scale_ref[...] = jnp.tanh(s / (pn + 1e-16)) * keepn

  return pl.pallas_call(
      body,
      out_shape=[
          jax.ShapeDtypeStruct((80, 128), _f32),
          jax.ShapeDtypeStruct((80, 128), _f32),
      ],
      name=f"topk_{k}",
  )


def _make_pool(kn):
  bm = 256
  nrow = NP // bm

  def body(h, scale, keep, x3_ref, ro_ref):
    i = pl.program_id(1)
    xn = h[...] * scale[...]
    x3_ref[...] = xn
    masked = jnp.where(keep[...] > 0.0, xn, -3.4e38)
    cmax = jnp.max(masked, axis=0, keepdims=True)
    csum = jnp.sum(xn, axis=0, keepdims=True)

    @pl.when(i == 0)
    def _():
      ro_ref[...] = jnp.concatenate([cmax, csum], axis=0)[None]

    @pl.when(i > 0)
    def _():
      cur = ro_ref[...]
      mx = jnp.maximum(cur[0, 0:1], cmax)
      sm = cur[0, 1:2] + csum
      ro_ref[...] = jnp.concatenate([mx, sm], axis=0)[None]

    @pl.when(i == nrow - 1)
    def _():
      cur = ro_ref[...]
      ro_ref[...] = jnp.concatenate(
          [cur[0, 0:1], cur[0, 1:2] * (1.0 / kn)], axis=0)[None]

  return pl.pallas_call(
      body,
      grid=(8, nrow),
      in_specs=[
          pl.BlockSpec((bm, F), lambda c, i: (i, c)),
          pl.BlockSpec((bm, 1), lambda c, i: (i, 0)),
          pl.BlockSpec((bm, 1), lambda c, i: (i, 0)),
      ],
      out_specs=[
          pl.BlockSpec((bm, F), lambda c, i: (c * nrow + i, 0)),
          pl.BlockSpec((1, 2, F), lambda c, i: (c, 0, 0)),
      ],
      out_shape=[
          jax.ShapeDtypeStruct((8 * NP, F), _f32),
          jax.ShapeDtypeStruct((8, 2, F), _f32),
      ],
      name=f"pool_{kn}",
  )


_edge_agg_1 = _make_edge_agg(1)
_edge_agg_8 = _make_edge_agg(8)
_revalidate = _make_revalidate()
_xr_d = _make_xr(D)
_xr_cm = _make_xr_cm()
_sage_d = _make_sage(D)
_sage_h = _make_sage(H)
_topk = {k: _make_topk(k) for k in (8000, 6400, 5120)}
_pool = {k: _make_pool(k) for k in (8000, 6400, 5120)}


def kernel(x, edge_index, batch, w1_l, w1_r, b1, p1, w2_l, w2_r, b2, p2,
           w3_l, w3_r, b3, p3):
  del batch
  xp = jnp.zeros((NP, D), _f32).at[:N].set(x)
  src = edge_index[0].astype(_i32)
  dst = edge_index[1].astype(_i32)
  npad = EP - E
  pad_dum = N + (jnp.arange(npad, dtype=_i32) % 32)
  srcp = jnp.concatenate([src, pad_dum])
  dstp = jnp.concatenate([dst, jnp.full((npad,), NP - 1, _i32)])
  valid = jnp.concatenate([jnp.ones((E,), _f32), jnp.zeros((npad,), _f32)])
  dst2 = dstp.reshape(EP // 128, 128)
  keep = jnp.concatenate([jnp.ones((N,), _f32), jnp.zeros((NP - N,), _f32)])

  srcef = srcp
  dstcur = dstp
  valtile = valid
  counts = jnp.full((32, 16), EPT, _i32)
  x3 = xp
  result = jnp.zeros((1, 2 * H), _f32)

  layers = [
      (w1_l, w1_r, b1, p1, _sage_d, _edge_agg_1, 8000),
      (w2_l, w2_r, b2, p2, _sage_h, _edge_agg_8, 6400),
      (w3_l, w3_r, b3, p3, _sage_h, _edge_agg_8, 5120),
  ]
  for li, (wl, wr, b, p, sage, eagg, kn) in enumerate(layers):
    if li == 0:
      xr = _xr_d(xp, wr, b.reshape(1, H))
    else:
      xr = _xr_cm(x3, wr, b.reshape(1, H))
    aggp, cntp = eagg(x3, srcef, dstcur.reshape(EP // 128, 128),
                      valtile.reshape(EP // 128, 128), counts)
    h, sraw = sage(aggp, cntp.reshape(2, NP, 1), xr, wl, p.reshape(H, 1))
    keepn2, scale2 = _topk[kn](sraw.reshape(80, 128), keep.reshape(80, 128),
                               p.reshape(8, 128))
    keepn = keepn2.reshape(NP)
    x3, ro = _pool[kn](h, scale2.reshape(NP, 1), keepn.reshape(NP, 1))
    result = result + jnp.concatenate(
        [ro[:, 0].reshape(1, H), ro[:, 1].reshape(1, H)], axis=1)
    if li < 2:
      srcef, dstcur, valtile, valid, counts = _revalidate(
          keepn, srcp, dstp, valid)
      keep = keepn
  return result


# speedup vs baseline: 1.0813x; 1.0713x over previous
import functools
import math

import jax
import jax.numpy as jnp
from jax import lax
from jax.experimental import pallas as pl
from jax.experimental.pallas import tpu as pltpu
from jax.experimental.pallas import tpu_sc as plsc

N = 10000
E = 160000
D = 128
H = 1024

NP = 10240
F = 128
EP = 163840
EPT = EP // 32
NBATCH = EPT // 128
NSTRIPE = NP // 16

_f32 = jnp.float32
_i32 = jnp.int32


def _make_edge_agg(nch):
  mesh = plsc.VectorSubcoreMesh(core_axis_name="c", subcore_axis_name="s")

  def body(x3, srcef, dst2, val2, counts, aggp, cntp,
           aggsp, cntsp, idxf, idxb, dstst, valst, gbuf, cbuf, ctile, sem):
    cid = lax.axis_index("c")
    sid = lax.axis_index("s")
    tid = cid * 16 + sid
    ebase = pl.multiple_of(cid * (EP // 2) + sid * EPT, EPT)
    rbase = pl.multiple_of(ebase // 128, NBATCH)
    r0 = pl.multiple_of(sid * NSTRIPE, NSTRIPE)

    pltpu.sync_copy(srcef.at[pl.ds(ebase, EPT)], idxf)
    pltpu.sync_copy(dst2.at[pl.ds(rbase, NBATCH)], dstst)
    pltpu.sync_copy(val2.at[pl.ds(rbase, NBATCH)], valst)
    pltpu.sync_copy(counts.at[tid], ctile)
    nb = (ctile[...][0] + 127) // 128

    for ch in range(nch):
      def zfill(t, carry):
        gbuf[t // 8, pl.ds((t % 8) * 16, 16)] = jnp.zeros((16,), _f32)
        return carry
      lax.fori_loop(0, 128 * 8, zfill, 0)
      for m in range(NSTRIPE // 128):
        pltpu.sync_copy(gbuf, aggsp.at[pl.ds(r0 + m * 128, 128)])
      if ch == 0:
        for m in range(NSTRIPE // 128):
          pltpu.sync_copy(gbuf.at[0], cntsp.at[pl.ds(r0 + m * 128, 128)])
      plsc.subcore_barrier()

      coff = ch * NP

      def batch(j, carry):
        if nch > 1:
          def afill(t, carry2):
            idxb[pl.ds(t * 16, 16)] = idxf[pl.ds(j * 128 + t * 16, 16)] + coff
            return carry2
          lax.fori_loop(0, 8, afill, 0)
          idxsrc = idxb
        else:
          idxsrc = idxf.at[pl.ds(j * 128, 128)]
        pltpu.async_copy(x3.at[idxsrc], gbuf, sem).wait()
        pltpu.sync_copy(gbuf, aggsp.at[dstst.at[j]], add=True)
        return carry
      lax.fori_loop(0, nb, batch, 0)

      if ch == 0:
        def cbatch(j, carry):
          pltpu.sync_copy(valst.at[j], cntsp.at[dstst.at[j]], add=True)
          return carry
        lax.fori_loop(0, nb, cbatch, 0)

      plsc.subcore_barrier()

      for m in range(NSTRIPE // 128):
        pltpu.sync_copy(aggsp.at[pl.ds(r0 + m * 128, 128)], gbuf)
        pltpu.sync_copy(
            gbuf, aggp.at[cid, pl.ds(r0 + m * 128, 128), pl.ds(ch * F, F)])
      if ch == 0:
        pltpu.sync_copy(cntsp.at[pl.ds(r0, NSTRIPE)], cbuf)
        pltpu.sync_copy(cbuf, cntp.at[cid, pl.ds(r0, NSTRIPE)])

  return pl.kernel(
      body,
      out_type=[
          jax.ShapeDtypeStruct((2, NP, nch * F), _f32),
          jax.ShapeDtypeStruct((2, NP), _f32),
      ],
      mesh=mesh,
      scratch_types=[
          pltpu.VMEM_SHARED((NP, F), _f32),
          pltpu.VMEM_SHARED((NP,), _f32),
          pltpu.VMEM((EPT,), _i32),
          pltpu.VMEM((128,), _i32),
          pltpu.VMEM((NBATCH, 128), _i32),
          pltpu.VMEM((NBATCH, 128), _f32),
          pltpu.VMEM((128, F), _f32),
          pltpu.VMEM((NSTRIPE,), _f32),
          pltpu.VMEM((16,), _i32),
          pltpu.SemaphoreType.DMA,
      ],
      name=f"edge_agg_{nch}",
  )


def _make_revalidate():
  mesh = plsc.VectorSubcoreMesh(core_axis_name="c", subcore_axis_name="s")

  def body(keep, srcp, dstp, valp, srcc_o, dstc_o, valc_o, valn_o, cnts_o,
           keepst, sst, dstt, vst_, sout, dout, vout, vfull, cb16):
    cid = lax.axis_index("c")
    sid = lax.axis_index("s")
    tid = cid * 16 + sid
    ebase = pl.multiple_of(cid * (EP // 2) + sid * EPT, EPT)

    pltpu.sync_copy(keep, keepst)
    pltpu.sync_copy(srcp.at[pl.ds(ebase, EPT)], sst)
    pltpu.sync_copy(dstp.at[pl.ds(ebase, EPT)], dstt)
    pltpu.sync_copy(valp.at[pl.ds(ebase, EPT)], vst_)

    iota16 = lax.iota(_i32, 16)
    zeros16i = jnp.zeros((16,), _i32)
    ones16 = jnp.ones((16,), _f32)

    def pre(t, c):
      sout[pl.ds(t * 16, 16)] = N + ((iota16 + t) % 32)
      dout[pl.ds(t * 16, 16)] = zeros16i + (NP - 1)
      vout[pl.ds(t * 16, 16)] = jnp.zeros((16,), _f32)
      return c
    lax.fori_loop(0, EPT // 16, pre, 0)

    def step(t, cnt):
      s16 = sst[pl.ds(t * 16, 16)]
      d16 = dstt[pl.ds(t * 16, 16)]
      v16 = vst_[pl.ds(t * 16, 16)]
      ks = plsc.load_gather(keepst, [s16])
      kd = plsc.load_gather(keepst, [d16])
      v = v16 * ks * kd
      vfull[pl.ds(t * 16, 16)] = v
      m = v > 0.0
      vi = jnp.where(m, 1, 0)
      pos = plsc.cumsum(vi) + (cnt - 1)
      plsc.store_scatter(sout, [pos], s16, mask=m)
      plsc.store_scatter(dout, [pos], d16, mask=m)
      plsc.store_scatter(vout, [pos], ones16, mask=m)
      return cnt + jnp.sum(vi)
    cnt = lax.fori_loop(0, EPT // 16, step, 0)

    cb16[...] = zeros16i + cnt
    pltpu.sync_copy(sout, srcc_o.at[pl.ds(ebase, EPT)])
    pltpu.sync_copy(dout, dstc_o.at[pl.ds(ebase, EPT)])
    pltpu.sync_copy(vout, valc_o.at[pl.ds(ebase, EPT)])
    pltpu.sync_copy(vfull, valn_o.at[pl.ds(ebase, EPT)])
    pltpu.sync_copy(cb16, cnts_o.at[tid])

  return pl.kernel(
      body,
      out_type=[
          jax.ShapeDtypeStruct((EP,), _i32),
          jax.ShapeDtypeStruct((EP,), _i32),
          jax.ShapeDtypeStruct((EP,), _f32),
          jax.ShapeDtypeStruct((EP,), _f32),
          jax.ShapeDtypeStruct((32, 16), _i32),
      ],
      mesh=mesh,
      scratch_types=[
          pltpu.VMEM((NP,), _f32),
          pltpu.VMEM((EPT,), _i32),
          pltpu.VMEM((EPT,), _i32),
          pltpu.VMEM((EPT,), _f32),
          pltpu.VMEM((EPT,), _i32),
          pltpu.VMEM((EPT,), _i32),
          pltpu.VMEM((EPT,), _f32),
          pltpu.VMEM((EPT,), _f32),
          pltpu.VMEM((16,), _i32),
      ],
      compiler_params=pltpu.CompilerParams(needs_layout_passes=False),
      name="revalidate",
  )


def _make_xr(din):
  bm = 256

  def body(x, wr, b, xr_ref):
    xr_ref[...] = (jnp.dot(x[...], wr[...], preferred_element_type=_f32)
                   + b[...])

  return pl.pallas_call(
      body,
      grid=(NP // bm,),
      in_specs=[
          pl.BlockSpec((bm, din), lambda i: (i, 0)),
          pl.BlockSpec((din, H), lambda i: (0, 0)),
          pl.BlockSpec((1, H), lambda i: (0, 0)),
      ],
      out_specs=pl.BlockSpec((bm, H), lambda i: (i, 0)),
      out_shape=jax.ShapeDtypeStruct((NP, H), _f32),
      name=f"xr_{din}",
  )


def _make_xr_cm():
  bm = 256
  nrow = NP // bm

  def body(x3, wr, b, xr_ref):
    c = pl.program_id(1)

    @pl.when(c == 0)
    def _():
      xr_ref[...] = jnp.zeros((bm, H), _f32) + b[...]

    xr_ref[...] += jnp.dot(x3[...], wr[...], preferred_element_type=_f32)

  return pl.pallas_call(
      body,
      grid=(nrow, 8),
      in_specs=[
          pl.BlockSpec((bm, F), lambda i, c: (c * nrow + i, 0)),
          pl.BlockSpec((F, H), lambda i, c: (c, 0)),
          pl.BlockSpec((1, H), lambda i, c: (0, 0)),
      ],
      out_specs=pl.BlockSpec((bm, H), lambda i, c: (i, 0)),
      out_shape=jax.ShapeDtypeStruct((NP, H), _f32),
      name="xr_cm",
  )


def _make_sage(din):
  bm = 256

  def body(aggp, cntp, xr, wl, p, h_ref, sraw_ref):
    agg = aggp[0] + aggp[1]
    c = cntp[0] + cntp[1]
    mean = jnp.where(c > 0.0, agg / jnp.maximum(c, 1.0), 0.0)
    hm = jnp.dot(mean, wl[...], preferred_element_type=_f32) + xr[...]
    h = jnp.maximum(hm, 0.0)
    h_ref[...] = h
    sraw_ref[...] = jnp.dot(h, p[...], preferred_element_type=_f32)

  return pl.pallas_call(
      body,
      grid=(NP // bm,),
      in_specs=[
          pl.BlockSpec((2, bm, din), lambda i: (0, i, 0)),
          pl.BlockSpec((2, bm, 1), lambda i: (0, i, 0)),
          pl.BlockSpec((bm, H), lambda i: (i, 0)),
          pl.BlockSpec((din, H), lambda i: (0, 0)),
          pl.BlockSpec((H, 1), lambda i: (0, 0)),
      ],
      out_specs=[
          pl.BlockSpec((bm, H), lambda i: (i, 0)),
          pl.BlockSpec((bm, 1), lambda i: (i, 0)),
      ],
      out_shape=[
          jax.ShapeDtypeStruct((NP, H), _f32),
          jax.ShapeDtypeStruct((NP, 1), _f32),
      ],
      name=f"sage_{din}",
  )


def _make_topk(k):
  def body(sraw, keep, p, keepn_ref, scale_ref):
    s = sraw[...]
    kp = keep[...]
    bits = lax.bitcast_convert_type(s, jnp.uint32)
    top = jnp.uint32(0x80000000)
    sortable = jnp.where((bits & top) != 0, ~bits, bits | top)
    hi = jnp.where(kp > 0.0, sortable, jnp.uint32(0))
    ridx = (lax.broadcasted_iota(_i32, (80, 128), 0) * 128
            + lax.broadcasted_iota(_i32, (80, 128), 1))
    lo = (NP - ridx).astype(jnp.uint32)

    def hstep(t, pref):
      cand = pref | (jnp.uint32(1) << (31 - t).astype(jnp.uint32))
      cnt = jnp.sum((hi >= cand).astype(_i32))
      return jnp.where(cnt >= k, cand, pref)
    hstar = lax.fori_loop(0, 32, hstep, jnp.uint32(0))

    ngt = jnp.sum((hi > hstar).astype(_i32))
    r = k - ngt
    tie = hi == hstar

    def lstep(t, pref):
      cand = pref | (jnp.uint32(1) << (13 - t).astype(jnp.uint32))
      cnt = jnp.sum((tie & (lo >= cand)).astype(_i32))
      return jnp.where(cnt >= r, cand, pref)
    lstar = lax.fori_loop(0, 14, lstep, jnp.uint32(0))

    keepn = ((hi > hstar) | (tie & (lo >= lstar))).astype(_f32)
    keepn_ref[...] = keepn
    pn = jnp.sqrt(jnp.sum(p[...] * p[...]))
    scale_ref[...] = jnp.tanh(s / (pn + 1e-16)) * keepn

  return pl.pallas_call(
      body,
      out_shape=[
          jax.ShapeDtypeStruct((80, 128), _f32),
          jax.ShapeDtypeStruct((80, 128), _f32),
      ],
      name=f"topk_{k}",
  )


def _make_pool(kn):
  bm = 256
  nrow = NP // bm

  def body(h, scale, keep, x3_ref, ro_ref):
    i = pl.program_id(0)
    c = pl.program_id(1)
    xn = h[:, pl.ds(c * F, F)] * scale[...]
    x3_ref[...] = xn
    masked = jnp.where(keep[...] > 0.0, xn, -3.4e38)
    cmax = jnp.max(masked, axis=0, keepdims=True)
    csum = jnp.sum(xn, axis=0, keepdims=True)
    cur = jnp.concatenate([cmax, csum], axis=0)[None]

    @pl.when(i == 0)
    def _():
      ro_ref[pl.ds(c, 1)] = cur

    @pl.when(i > 0)
    def _():
      prev = ro_ref[pl.ds(c, 1)]
      mx = jnp.maximum(prev[0, 0:1], cmax)
      sm = prev[0, 1:2] + csum
      ro_ref[pl.ds(c, 1)] = jnp.concatenate([mx, sm], axis=0)[None]

    @pl.when(i == nrow - 1)
    def _():
      prev = ro_ref[pl.ds(c, 1)]
      ro_ref[pl.ds(c, 1)] = jnp.concatenate(
          [prev[0, 0:1], prev[0, 1:2] * (1.0 / kn)], axis=0)[None]

  return pl.pallas_call(
      body,
      grid=(nrow, 8),
      in_specs=[
          pl.BlockSpec((bm, H), lambda i, c: (i, 0)),
          pl.BlockSpec((bm, 1), lambda i, c: (i, 0)),
          pl.BlockSpec((bm, 1), lambda i, c: (i, 0)),
      ],
      out_specs=[
          pl.BlockSpec((bm, F), lambda i, c: (c * nrow + i, 0)),
          pl.BlockSpec((8, 2, F), lambda i, c: (0, 0, 0)),
      ],
      out_shape=[
          jax.ShapeDtypeStruct((8 * NP, F), _f32),
          jax.ShapeDtypeStruct((8, 2, F), _f32),
      ],
      name=f"pool_{kn}",
  )


_edge_agg_1 = _make_edge_agg(1)
_edge_agg_8 = _make_edge_agg(8)
_revalidate = _make_revalidate()
_xr_d = _make_xr(D)
_xr_cm = _make_xr_cm()
_sage_d = _make_sage(D)
_sage_h = _make_sage(H)
_topk = {k: _make_topk(k) for k in (8000, 6400, 5120)}
_pool = {k: _make_pool(k) for k in (8000, 6400, 5120)}


def kernel(x, edge_index, batch, w1_l, w1_r, b1, p1, w2_l, w2_r, b2, p2,
           w3_l, w3_r, b3, p3):
  del batch
  xp = jnp.zeros((NP, D), _f32).at[:N].set(x)
  src = edge_index[0].astype(_i32)
  dst = edge_index[1].astype(_i32)
  npad = EP - E
  pad_dum = N + (jnp.arange(npad, dtype=_i32) % 32)
  srcp = jnp.concatenate([src, pad_dum])
  dstp = jnp.concatenate([dst, jnp.full((npad,), NP - 1, _i32)])
  valid = jnp.concatenate([jnp.ones((E,), _f32), jnp.zeros((npad,), _f32)])
  dst2 = dstp.reshape(EP // 128, 128)
  keep = jnp.concatenate([jnp.ones((N,), _f32), jnp.zeros((NP - N,), _f32)])

  srcef = srcp
  dstcur = dstp
  valtile = valid
  counts = jnp.full((32, 16), EPT, _i32)
  x3 = xp
  result = jnp.zeros((1, 2 * H), _f32)

  layers = [
      (w1_l, w1_r, b1, p1, _sage_d, _edge_agg_1, 8000),
      (w2_l, w2_r, b2, p2, _sage_h, _edge_agg_8, 6400),
      (w3_l, w3_r, b3, p3, _sage_h, _edge_agg_8, 5120),
  ]
  for li, (wl, wr, b, p, sage, eagg, kn) in enumerate(layers):
    if li == 0:
      xr = _xr_d(xp, wr, b.reshape(1, H))
    else:
      xr = _xr_cm(x3, wr, b.reshape(1, H))
    aggp, cntp = eagg(x3, srcef, dstcur.reshape(EP // 128, 128),
                      valtile.reshape(EP // 128, 128), counts)
    h, sraw = sage(aggp, cntp.reshape(2, NP, 1), xr, wl, p.reshape(H, 1))
    keepn2, scale2 = _topk[kn](sraw.reshape(80, 128), keep.reshape(80, 128),
                               p.reshape(8, 128))
    keepn = keepn2.reshape(NP)
    x3, ro = _pool[kn](h, scale2.reshape(NP, 1), keepn.reshape(NP, 1))
    result = result + jnp.concatenate(
        [ro[:, 0].reshape(1, H), ro[:, 1].reshape(1, H)], axis=1)
    if li < 2:
      srcef, dstcur, valtile, valid, counts = _revalidate(
          keepn, srcp, dstp, valid)
      keep = keepn
  return result


# speedup vs baseline: 1.0887x; 1.0069x over previous
import functools
import math

import jax
import jax.numpy as jnp
from jax import lax
from jax.experimental import pallas as pl
from jax.experimental.pallas import tpu as pltpu
from jax.experimental.pallas import tpu_sc as plsc

N = 10000
E = 160000
D = 128
H = 1024

NP = 10240
F = 128
EP = 163840
EPT = EP // 32
BB = 256
NSTRIPE = NP // 16

_f32 = jnp.float32
_i32 = jnp.int32


def _make_edge_agg(nch):
  mesh = plsc.VectorSubcoreMesh(core_axis_name="c", subcore_axis_name="s")

  def body(x3, srcef, dstf, counts, aggp, cntp,
           aggsp, cntsp, idxf, idxb, dstst, vones, gbuf, cbuf, ctile, sem):
    cid = lax.axis_index("c")
    sid = lax.axis_index("s")
    tid = cid * 16 + sid
    ebase = pl.multiple_of(cid * (EP // 2) + sid * EPT, EPT)
    r0 = pl.multiple_of(sid * NSTRIPE, NSTRIPE)

    pltpu.sync_copy(srcef.at[pl.ds(ebase, EPT)], idxf)
    pltpu.sync_copy(dstf.at[pl.ds(ebase, EPT)], dstst)
    pltpu.sync_copy(counts.at[tid], ctile)
    nb = (ctile[...][0] + (BB - 1)) // BB

    def ofill(t, carry):
      vones[pl.ds(t * 16, 16)] = jnp.ones((16,), _f32)
      return carry
    lax.fori_loop(0, BB // 16, ofill, 0)

    for ch in range(nch):
      def zfill(t, carry):
        gbuf[t // 8, pl.ds((t % 8) * 16, 16)] = jnp.zeros((16,), _f32)
        return carry
      lax.fori_loop(0, 128 * 8, zfill, 0)
      for m in range(NSTRIPE // 128):
        pltpu.sync_copy(gbuf.at[pl.ds(0, 128)], aggsp.at[pl.ds(r0 + m * 128, 128)])
      if ch == 0:
        for m in range(NSTRIPE // 128):
          pltpu.sync_copy(gbuf.at[0], cntsp.at[pl.ds(r0 + m * 128, 128)])
      plsc.subcore_barrier()

      coff = ch * NP

      def batch(j, carry):
        if nch > 1:
          def afill(t, carry2):
            idxb[pl.ds(t * 16, 16)] = idxf[pl.ds(j * BB + t * 16, 16)] + coff
            return carry2
          lax.fori_loop(0, BB // 16, afill, 0)
          idxsrc = idxb
        else:
          idxsrc = idxf.at[pl.ds(j * BB, BB)]
        pltpu.async_copy(x3.at[idxsrc], gbuf, sem).wait()
        pltpu.sync_copy(gbuf, aggsp.at[dstst.at[pl.ds(j * BB, BB)]], add=True)
        return carry
      lax.fori_loop(0, nb, batch, 0)

      if ch == 0:
        def cbatch(j, carry):
          pltpu.sync_copy(vones,
                          cntsp.at[dstst.at[pl.ds(j * BB, BB)]], add=True)
          return carry
        lax.fori_loop(0, nb, cbatch, 0)

      plsc.subcore_barrier()

      for m in range(NSTRIPE // 128):
        pltpu.sync_copy(aggsp.at[pl.ds(r0 + m * 128, 128)], gbuf.at[pl.ds(0, 128)])
        pltpu.sync_copy(
            gbuf.at[pl.ds(0, 128)],
            aggp.at[cid, pl.ds(r0 + m * 128, 128), pl.ds(ch * F, F)])
      if ch == 0:
        pltpu.sync_copy(cntsp.at[pl.ds(r0, NSTRIPE)], cbuf)
        pltpu.sync_copy(cbuf, cntp.at[cid, pl.ds(r0, NSTRIPE)])

  return pl.kernel(
      body,
      out_type=[
          jax.ShapeDtypeStruct((2, NP, nch * F), _f32),
          jax.ShapeDtypeStruct((2, NP), _f32),
      ],
      mesh=mesh,
      scratch_types=[
          pltpu.VMEM_SHARED((NP, F), _f32),
          pltpu.VMEM_SHARED((NP,), _f32),
          pltpu.VMEM((EPT,), _i32),
          pltpu.VMEM((BB,), _i32),
          pltpu.VMEM((EPT,), _i32),
          pltpu.VMEM((BB,), _f32),
          pltpu.VMEM((BB, F), _f32),
          pltpu.VMEM((NSTRIPE,), _f32),
          pltpu.VMEM((16,), _i32),
          pltpu.SemaphoreType.DMA,
      ],
      name=f"edge_agg_{nch}",
  )


def _make_revalidate():
  mesh = plsc.VectorSubcoreMesh(core_axis_name="c", subcore_axis_name="s")

  def body(keep, srcp, dstp, valp, srcc_o, dstc_o, valc_o, valn_o, cnts_o,
           keepst, sst, dstt, vst_, sout, dout, vout, vfull, cb16):
    cid = lax.axis_index("c")
    sid = lax.axis_index("s")
    tid = cid * 16 + sid
    ebase = pl.multiple_of(cid * (EP // 2) + sid * EPT, EPT)

    pltpu.sync_copy(keep, keepst)
    pltpu.sync_copy(srcp.at[pl.ds(ebase, EPT)], sst)
    pltpu.sync_copy(dstp.at[pl.ds(ebase, EPT)], dstt)
    pltpu.sync_copy(valp.at[pl.ds(ebase, EPT)], vst_)

    iota16 = lax.iota(_i32, 16)
    zeros16i = jnp.zeros((16,), _i32)
    ones16 = jnp.ones((16,), _f32)

    def pre(t, c):
      sout[pl.ds(t * 16, 16)] = N + ((iota16 + t) % 32)
      dout[pl.ds(t * 16, 16)] = zeros16i + (NP - 1)
      vout[pl.ds(t * 16, 16)] = jnp.zeros((16,), _f32)
      return c
    lax.fori_loop(0, EPT // 16, pre, 0)

    def step(t, cnt):
      s16 = sst[pl.ds(t * 16, 16)]
      d16 = dstt[pl.ds(t * 16, 16)]
      v16 = vst_[pl.ds(t * 16, 16)]
      ks = plsc.load_gather(keepst, [s16])
      kd = plsc.load_gather(keepst, [d16])
      v = v16 * ks * kd
      vfull[pl.ds(t * 16, 16)] = v
      m = v > 0.0
      vi = jnp.where(m, 1, 0)
      pos = plsc.cumsum(vi) + (cnt - 1)
      plsc.store_scatter(sout, [pos], s16, mask=m)
      plsc.store_scatter(dout, [pos], d16, mask=m)
      plsc.store_scatter(vout, [pos], ones16, mask=m)
      return cnt + jnp.sum(vi)
    cnt = lax.fori_loop(0, EPT // 16, step, 0)

    cb16[...] = zeros16i + cnt
    pltpu.sync_copy(sout, srcc_o.at[pl.ds(ebase, EPT)])
    pltpu.sync_copy(dout, dstc_o.at[pl.ds(ebase, EPT)])
    pltpu.sync_copy(vout, valc_o.at[pl.ds(ebase, EPT)])
    pltpu.sync_copy(vfull, valn_o.at[pl.ds(ebase, EPT)])
    pltpu.sync_copy(cb16, cnts_o.at[tid])

  return pl.kernel(
      body,
      out_type=[
          jax.ShapeDtypeStruct((EP,), _i32),
          jax.ShapeDtypeStruct((EP,), _i32),
          jax.ShapeDtypeStruct((EP,), _f32),
          jax.ShapeDtypeStruct((EP,), _f32),
          jax.ShapeDtypeStruct((32, 16), _i32),
      ],
      mesh=mesh,
      scratch_types=[
          pltpu.VMEM((NP,), _f32),
          pltpu.VMEM((EPT,), _i32),
          pltpu.VMEM((EPT,), _i32),
          pltpu.VMEM((EPT,), _f32),
          pltpu.VMEM((EPT,), _i32),
          pltpu.VMEM((EPT,), _i32),
          pltpu.VMEM((EPT,), _f32),
          pltpu.VMEM((EPT,), _f32),
          pltpu.VMEM((16,), _i32),
      ],
      compiler_params=pltpu.CompilerParams(needs_layout_passes=False),
      name="revalidate",
  )


def _make_xr(din):
  bm = 256

  def body(x, wr, b, xr_ref):
    xr_ref[...] = (jnp.dot(x[...], wr[...], preferred_element_type=_f32)
                   + b[...])

  return pl.pallas_call(
      body,
      grid=(NP // bm,),
      in_specs=[
          pl.BlockSpec((bm, din), lambda i: (i, 0)),
          pl.BlockSpec((din, H), lambda i: (0, 0)),
          pl.BlockSpec((1, H), lambda i: (0, 0)),
      ],
      out_specs=pl.BlockSpec((bm, H), lambda i: (i, 0)),
      out_shape=jax.ShapeDtypeStruct((NP, H), _f32),
      name=f"xr_{din}",
  )


def _make_xr_cm():
  bm = 256
  nrow = NP // bm

  def body(x3, wr, b, xr_ref):
    c = pl.program_id(1)

    @pl.when(c == 0)
    def _():
      xr_ref[...] = jnp.zeros((bm, H), _f32) + b[...]

    xr_ref[...] += jnp.dot(x3[...], wr[...], preferred_element_type=_f32)

  return pl.pallas_call(
      body,
      grid=(nrow, 8),
      in_specs=[
          pl.BlockSpec((bm, F), lambda i, c: (c * nrow + i, 0)),
          pl.BlockSpec((F, H), lambda i, c: (c, 0)),
          pl.BlockSpec((1, H), lambda i, c: (0, 0)),
      ],
      out_specs=pl.BlockSpec((bm, H), lambda i, c: (i, 0)),
      out_shape=jax.ShapeDtypeStruct((NP, H), _f32),
      name="xr_cm",
  )


def _make_sage(din):
  bm = 256

  def body(aggp, cntp, xr, wl, p, h_ref, sraw_ref):
    agg = aggp[0] + aggp[1]
    c = cntp[0] + cntp[1]
    mean = jnp.where(c > 0.0, agg / jnp.maximum(c, 1.0), 0.0)
    hm = jnp.dot(mean, wl[...], preferred_element_type=_f32) + xr[...]
    h = jnp.maximum(hm, 0.0)
    h_ref[...] = h
    sraw_ref[...] = jnp.dot(h, p[...], preferred_element_type=_f32)

  return pl.pallas_call(
      body,
      grid=(NP // bm,),
      in_specs=[
          pl.BlockSpec((2, bm, din), lambda i: (0, i, 0)),
          pl.BlockSpec((2, bm, 1), lambda i: (0, i, 0)),
          pl.BlockSpec((bm, H), lambda i: (i, 0)),
          pl.BlockSpec((din, H), lambda i: (0, 0)),
          pl.BlockSpec((H, 1), lambda i: (0, 0)),
      ],
      out_specs=[
          pl.BlockSpec((bm, H), lambda i: (i, 0)),
          pl.BlockSpec((bm, 1), lambda i: (i, 0)),
      ],
      out_shape=[
          jax.ShapeDtypeStruct((NP, H), _f32),
          jax.ShapeDtypeStruct((NP, 1), _f32),
      ],
      name=f"sage_{din}",
  )


def _make_topk(k):
  def body(sraw, keep, p, keepn_ref, scale_ref):
    s = sraw[...]
    kp = keep[...]
    bits = lax.bitcast_convert_type(s, jnp.uint32)
    top = jnp.uint32(0x80000000)
    sortable = jnp.where((bits & top) != 0, ~bits, bits | top)
    hi = jnp.where(kp > 0.0, sortable, jnp.uint32(0))
    ridx = (lax.broadcasted_iota(_i32, (80, 128), 0) * 128
            + lax.broadcasted_iota(_i32, (80, 128), 1))
    lo = (NP - ridx).astype(jnp.uint32)

    def hstep(t, pref):
      cand = pref | (jnp.uint32(1) << (31 - t).astype(jnp.uint32))
      cnt = jnp.sum((hi >= cand).astype(_i32))
      return jnp.where(cnt >= k, cand, pref)
    hstar = lax.fori_loop(0, 32, hstep, jnp.uint32(0))

    ngt = jnp.sum((hi > hstar).astype(_i32))
    r = k - ngt
    tie = hi == hstar

    def lstep(t, pref):
      cand = pref | (jnp.uint32(1) << (13 - t).astype(jnp.uint32))
      cnt = jnp.sum((tie & (lo >= cand)).astype(_i32))
      return jnp.where(cnt >= r, cand, pref)
    lstar = lax.fori_loop(0, 14, lstep, jnp.uint32(0))

    keepn = ((hi > hstar) | (tie & (lo >= lstar))).astype(_f32)
    keepn_ref[...] = keepn
    pn = jnp.sqrt(jnp.sum(p[...] * p[...]))
    scale_ref[...] = jnp.tanh(s / (pn + 1e-16)) * keepn

  return pl.pallas_call(
      body,
      out_shape=[
          jax.ShapeDtypeStruct((80, 128), _f32),
          jax.ShapeDtypeStruct((80, 128), _f32),
      ],
      name=f"topk_{k}",
  )


def _make_pool(kn):
  bm = 256
  nrow = NP // bm

  def body(h, scale, keep, x3_ref, ro_ref):
    i = pl.program_id(0)
    c = pl.program_id(1)
    xn = h[:, pl.ds(c * F, F)] * scale[...]
    x3_ref[...] = xn
    masked = jnp.where(keep[...] > 0.0, xn, -3.4e38)
    cmax = jnp.max(masked, axis=0, keepdims=True)
    csum = jnp.sum(xn, axis=0, keepdims=True)
    cur = jnp.concatenate([cmax, csum], axis=0)[None]

    @pl.when(i == 0)
    def _():
      ro_ref[pl.ds(c, 1)] = cur

    @pl.when(i > 0)
    def _():
      prev = ro_ref[pl.ds(c, 1)]
      mx = jnp.maximum(prev[0, 0:1], cmax)
      sm = prev[0, 1:2] + csum
      ro_ref[pl.ds(c, 1)] = jnp.concatenate([mx, sm], axis=0)[None]

    @pl.when(i == nrow - 1)
    def _():
      prev = ro_ref[pl.ds(c, 1)]
      ro_ref[pl.ds(c, 1)] = jnp.concatenate(
          [prev[0, 0:1], prev[0, 1:2] * (1.0 / kn)], axis=0)[None]

  return pl.pallas_call(
      body,
      grid=(nrow, 8),
      in_specs=[
          pl.BlockSpec((bm, H), lambda i, c: (i, 0)),
          pl.BlockSpec((bm, 1), lambda i, c: (i, 0)),
          pl.BlockSpec((bm, 1), lambda i, c: (i, 0)),
      ],
      out_specs=[
          pl.BlockSpec((bm, F), lambda i, c: (c * nrow + i, 0)),
          pl.BlockSpec((8, 2, F), lambda i, c: (0, 0, 0)),
      ],
      out_shape=[
          jax.ShapeDtypeStruct((8 * NP, F), _f32),
          jax.ShapeDtypeStruct((8, 2, F), _f32),
      ],
      name=f"pool_{kn}",
  )


_edge_agg_1 = _make_edge_agg(1)
_edge_agg_8 = _make_edge_agg(8)
_revalidate = _make_revalidate()
_xr_d = _make_xr(D)
_xr_cm = _make_xr_cm()
_sage_d = _make_sage(D)
_sage_h = _make_sage(H)
_topk = {k: _make_topk(k) for k in (8000, 6400, 5120)}
_pool = {k: _make_pool(k) for k in (8000, 6400, 5120)}


def kernel(x, edge_index, batch, w1_l, w1_r, b1, p1, w2_l, w2_r, b2, p2,
           w3_l, w3_r, b3, p3):
  del batch
  xp = jnp.zeros((NP, D), _f32).at[:N].set(x)
  src = edge_index[0].astype(_i32)
  dst = edge_index[1].astype(_i32)
  npad = EP - E
  pad_dum = N + (jnp.arange(npad, dtype=_i32) % 32)
  srcp = jnp.concatenate([src, pad_dum])
  dstp = jnp.concatenate([dst, jnp.full((npad,), NP - 1, _i32)])
  valid = jnp.concatenate([jnp.ones((E,), _f32), jnp.zeros((npad,), _f32)])
  dst2 = dstp.reshape(EP // 128, 128)
  keep = jnp.concatenate([jnp.ones((N,), _f32), jnp.zeros((NP - N,), _f32)])

  srcef = srcp
  dstcur = dstp
  valtile = valid
  counts = jnp.full((32, 16), EPT, _i32)
  x3 = xp
  result = jnp.zeros((1, 2 * H), _f32)

  layers = [
      (w1_l, w1_r, b1, p1, _sage_d, _edge_agg_1, 8000),
      (w2_l, w2_r, b2, p2, _sage_h, _edge_agg_8, 6400),
      (w3_l, w3_r, b3, p3, _sage_h, _edge_agg_8, 5120),
  ]
  for li, (wl, wr, b, p, sage, eagg, kn) in enumerate(layers):
    if li == 0:
      xr = _xr_d(xp, wr, b.reshape(1, H))
    else:
      xr = _xr_cm(x3, wr, b.reshape(1, H))
    aggp, cntp = eagg(x3, srcef, dstcur, counts)
    h, sraw = sage(aggp, cntp.reshape(2, NP, 1), xr, wl, p.reshape(H, 1))
    keepn2, scale2 = _topk[kn](sraw.reshape(80, 128), keep.reshape(80, 128),
                               p.reshape(8, 128))
    keepn = keepn2.reshape(NP)
    x3, ro = _pool[kn](h, scale2.reshape(NP, 1), keepn.reshape(NP, 1))
    result = result + jnp.concatenate(
        [ro[:, 0].reshape(1, H), ro[:, 1].reshape(1, H)], axis=1)
    if li < 2:
      srcef, dstcur, valtile, valid, counts = _revalidate(
          keepn, srcp, dstp, valid)
      keep = keepn
  return result


# speedup vs baseline: 1.2902x; 1.1851x over previous
import functools
import math

import jax
import jax.numpy as jnp
from jax import lax
from jax.experimental import pallas as pl
from jax.experimental.pallas import tpu as pltpu
from jax.experimental.pallas import tpu_sc as plsc

N = 10000
E = 160000
D = 128
H = 1024

NP = 10240
F = 128
EP = 163840
EPT = EP // 32
BB = 128
NSTRIPE = NP // 16

_f32 = jnp.float32
_i32 = jnp.int32


def _make_edge_agg(nch):
  mesh = plsc.VectorSubcoreMesh(core_axis_name="c", subcore_axis_name="s")

  def body(x3, srcef, dstf, counts, aggp, cntp,
           aggsp, cntsp, idxf, idxb, dstst, vones, gbuf, cbuf, ctile, sem):
    cid = lax.axis_index("c")
    sid = lax.axis_index("s")
    tid = cid * 16 + sid
    ebase = pl.multiple_of(cid * (EP // 2) + sid * EPT, EPT)
    r0 = pl.multiple_of(sid * NSTRIPE, NSTRIPE)

    pltpu.sync_copy(srcef.at[pl.ds(ebase, EPT)], idxf)
    pltpu.sync_copy(dstf.at[pl.ds(ebase, EPT)], dstst)
    pltpu.sync_copy(counts.at[tid], ctile)
    nb = (ctile[...][0] + (BB - 1)) // BB

    def ofill(t, carry):
      vones[pl.ds(t * 16, 16)] = jnp.ones((16,), _f32)
      return carry
    lax.fori_loop(0, BB // 16, ofill, 0)

    for ch in range(nch):
      def zfill(t, carry):
        gbuf[0, t // 8, pl.ds((t % 8) * 16, 16)] = jnp.zeros((16,), _f32)
        return carry
      lax.fori_loop(0, 128 * 8, zfill, 0)
      for m in range(NSTRIPE // 128):
        pltpu.sync_copy(gbuf.at[0], aggsp.at[pl.ds(r0 + m * 128, 128)])
      if ch == 0:
        for m in range(NSTRIPE // 128):
          pltpu.sync_copy(gbuf.at[0, 0], cntsp.at[pl.ds(r0 + m * 128, 128)])
      plsc.subcore_barrier()

      coff = ch * NP

      def gsrc(j, par):
        if nch > 1:
          return x3.at[idxb.at[par]]
        return x3.at[idxf.at[pl.ds(j * BB, BB)]]

      def start(j):
        par = j % 2
        if nch > 1:
          def afill(t, carry2):
            idxb[par, pl.ds(t * 16, 16)] = (
                idxf[pl.ds(j * BB + t * 16, 16)] + coff)
            return carry2
          lax.fori_loop(0, BB // 16, afill, 0)
        pltpu.async_copy(gsrc(j, par), gbuf.at[par], sem.at[par])

      @pl.when(nb > 0)
      def _():
        start(0)

      def batch(j, carry):
        @pl.when(j + 1 < nb)
        def _():
          start(j + 1)
        par = j % 2
        pltpu.make_async_copy(gsrc(j, par), gbuf.at[par], sem.at[par]).wait()
        pltpu.sync_copy(gbuf.at[par],
                        aggsp.at[dstst.at[pl.ds(j * BB, BB)]], add=True)
        return carry
      lax.fori_loop(0, nb, batch, 0)

      if ch == 0:
        def cbatch(j, carry):
          pltpu.sync_copy(vones,
                          cntsp.at[dstst.at[pl.ds(j * BB, BB)]], add=True)
          return carry
        lax.fori_loop(0, nb, cbatch, 0)

      plsc.subcore_barrier()

      for m in range(NSTRIPE // 128):
        pltpu.sync_copy(aggsp.at[pl.ds(r0 + m * 128, 128)], gbuf.at[0])
        pltpu.sync_copy(
            gbuf.at[0],
            aggp.at[cid, pl.ds(r0 + m * 128, 128), pl.ds(ch * F, F)])
      if ch == 0:
        pltpu.sync_copy(cntsp.at[pl.ds(r0, NSTRIPE)], cbuf)
        pltpu.sync_copy(cbuf, cntp.at[cid, pl.ds(r0, NSTRIPE)])

  return pl.kernel(
      body,
      out_type=[
          jax.ShapeDtypeStruct((2, NP, nch * F), _f32),
          jax.ShapeDtypeStruct((2, NP), _f32),
      ],
      mesh=mesh,
      scratch_types=[
          pltpu.VMEM_SHARED((NP, F), _f32),
          pltpu.VMEM_SHARED((NP,), _f32),
          pltpu.VMEM((EPT,), _i32),
          pltpu.VMEM((2, BB), _i32),
          pltpu.VMEM((EPT,), _i32),
          pltpu.VMEM((BB,), _f32),
          pltpu.VMEM((2, BB, F), _f32),
          pltpu.VMEM((NSTRIPE,), _f32),
          pltpu.VMEM((16,), _i32),
          pltpu.SemaphoreType.DMA((2,)),
      ],
      name=f"edge_agg_{nch}",
  )


def _make_revalidate():
  mesh = plsc.VectorSubcoreMesh(core_axis_name="c", subcore_axis_name="s")

  def body(keep, srcp, dstp, valp, srcc_o, dstc_o, valc_o, valn_o, cnts_o,
           keepst, sst, dstt, vst_, sout, dout, vout, vfull, cb16):
    cid = lax.axis_index("c")
    sid = lax.axis_index("s")
    tid = cid * 16 + sid
    ebase = pl.multiple_of(cid * (EP // 2) + sid * EPT, EPT)

    pltpu.sync_copy(keep, keepst)
    pltpu.sync_copy(srcp.at[pl.ds(ebase, EPT)], sst)
    pltpu.sync_copy(dstp.at[pl.ds(ebase, EPT)], dstt)
    pltpu.sync_copy(valp.at[pl.ds(ebase, EPT)], vst_)

    iota16 = lax.iota(_i32, 16)
    zeros16i = jnp.zeros((16,), _i32)
    ones16 = jnp.ones((16,), _f32)

    def pre(t, c):
      sout[pl.ds(t * 16, 16)] = N + ((iota16 + t) % 32)
      dout[pl.ds(t * 16, 16)] = zeros16i + (NP - 1)
      vout[pl.ds(t * 16, 16)] = jnp.zeros((16,), _f32)
      return c
    lax.fori_loop(0, EPT // 16, pre, 0)

    def step(t, cnt):
      s16 = sst[pl.ds(t * 16, 16)]
      d16 = dstt[pl.ds(t * 16, 16)]
      v16 = vst_[pl.ds(t * 16, 16)]
      ks = plsc.load_gather(keepst, [s16])
      kd = plsc.load_gather(keepst, [d16])
      v = v16 * ks * kd
      vfull[pl.ds(t * 16, 16)] = v
      m = v > 0.0
      vi = jnp.where(m, 1, 0)
      pos = plsc.cumsum(vi) + (cnt - 1)
      plsc.store_scatter(sout, [pos], s16, mask=m)
      plsc.store_scatter(dout, [pos], d16, mask=m)
      plsc.store_scatter(vout, [pos], ones16, mask=m)
      return cnt + jnp.sum(vi)
    cnt = lax.fori_loop(0, EPT // 16, step, 0)

    cb16[...] = zeros16i + cnt
    pltpu.sync_copy(sout, srcc_o.at[pl.ds(ebase, EPT)])
    pltpu.sync_copy(dout, dstc_o.at[pl.ds(ebase, EPT)])
    pltpu.sync_copy(vout, valc_o.at[pl.ds(ebase, EPT)])
    pltpu.sync_copy(vfull, valn_o.at[pl.ds(ebase, EPT)])
    pltpu.sync_copy(cb16, cnts_o.at[tid])

  return pl.kernel(
      body,
      out_type=[
          jax.ShapeDtypeStruct((EP,), _i32),
          jax.ShapeDtypeStruct((EP,), _i32),
          jax.ShapeDtypeStruct((EP,), _f32),
          jax.ShapeDtypeStruct((EP,), _f32),
          jax.ShapeDtypeStruct((32, 16), _i32),
      ],
      mesh=mesh,
      scratch_types=[
          pltpu.VMEM((NP,), _f32),
          pltpu.VMEM((EPT,), _i32),
          pltpu.VMEM((EPT,), _i32),
          pltpu.VMEM((EPT,), _f32),
          pltpu.VMEM((EPT,), _i32),
          pltpu.VMEM((EPT,), _i32),
          pltpu.VMEM((EPT,), _f32),
          pltpu.VMEM((EPT,), _f32),
          pltpu.VMEM((16,), _i32),
      ],
      compiler_params=pltpu.CompilerParams(needs_layout_passes=False),
      name="revalidate",
  )


def _make_xr(din):
  bm = 256

  def body(x, wr, b, xr_ref):
    xr_ref[...] = (jnp.dot(x[...], wr[...], preferred_element_type=_f32)
                   + b[...])

  return pl.pallas_call(
      body,
      grid=(NP // bm,),
      in_specs=[
          pl.BlockSpec((bm, din), lambda i: (i, 0)),
          pl.BlockSpec((din, H), lambda i: (0, 0)),
          pl.BlockSpec((1, H), lambda i: (0, 0)),
      ],
      out_specs=pl.BlockSpec((bm, H), lambda i: (i, 0)),
      out_shape=jax.ShapeDtypeStruct((NP, H), _f32),
      name=f"xr_{din}",
  )


def _make_xr_cm():
  bm = 256
  nrow = NP // bm

  def body(x3, wr, b, xr_ref):
    c = pl.program_id(1)

    @pl.when(c == 0)
    def _():
      xr_ref[...] = jnp.zeros((bm, H), _f32) + b[...]

    xr_ref[...] += jnp.dot(x3[...], wr[...], preferred_element_type=_f32)

  return pl.pallas_call(
      body,
      grid=(nrow, 8),
      in_specs=[
          pl.BlockSpec((bm, F), lambda i, c: (c * nrow + i, 0)),
          pl.BlockSpec((F, H), lambda i, c: (c, 0)),
          pl.BlockSpec((1, H), lambda i, c: (0, 0)),
      ],
      out_specs=pl.BlockSpec((bm, H), lambda i, c: (i, 0)),
      out_shape=jax.ShapeDtypeStruct((NP, H), _f32),
      name="xr_cm",
  )


def _make_sage(din):
  bm = 256

  def body(aggp, cntp, xr, wl, p, h_ref, sraw_ref):
    agg = aggp[0] + aggp[1]
    c = cntp[0] + cntp[1]
    mean = jnp.where(c > 0.0, agg / jnp.maximum(c, 1.0), 0.0)
    hm = jnp.dot(mean, wl[...], preferred_element_type=_f32) + xr[...]
    h = jnp.maximum(hm, 0.0)
    h_ref[...] = h
    sraw_ref[...] = jnp.dot(h, p[...], preferred_element_type=_f32)

  return pl.pallas_call(
      body,
      grid=(NP // bm,),
      in_specs=[
          pl.BlockSpec((2, bm, din), lambda i: (0, i, 0)),
          pl.BlockSpec((2, bm, 1), lambda i: (0, i, 0)),
          pl.BlockSpec((bm, H), lambda i: (i, 0)),
          pl.BlockSpec((din, H), lambda i: (0, 0)),
          pl.BlockSpec((H, 1), lambda i: (0, 0)),
      ],
      out_specs=[
          pl.BlockSpec((bm, H), lambda i: (i, 0)),
          pl.BlockSpec((bm, 1), lambda i: (i, 0)),
      ],
      out_shape=[
          jax.ShapeDtypeStruct((NP, H), _f32),
          jax.ShapeDtypeStruct((NP, 1), _f32),
      ],
      name=f"sage_{din}",
  )


def _make_topk(k):
  def body(sraw, keep, p, keepn_ref, scale_ref):
    s = sraw[...]
    kp = keep[...]
    bits = lax.bitcast_convert_type(s, jnp.uint32)
    top = jnp.uint32(0x80000000)
    sortable = jnp.where((bits & top) != 0, ~bits, bits | top)
    hi = jnp.where(kp > 0.0, sortable, jnp.uint32(0))
    ridx = (lax.broadcasted_iota(_i32, (80, 128), 0) * 128
            + lax.broadcasted_iota(_i32, (80, 128), 1))
    lo = (NP - ridx).astype(jnp.uint32)

    def hstep(t, pref):
      cand = pref | (jnp.uint32(1) << (31 - t).astype(jnp.uint32))
      cnt = jnp.sum((hi >= cand).astype(_i32))
      return jnp.where(cnt >= k, cand, pref)
    hstar = lax.fori_loop(0, 32, hstep, jnp.uint32(0))

    ngt = jnp.sum((hi > hstar).astype(_i32))
    r = k - ngt
    tie = hi == hstar

    def lstep(t, pref):
      cand = pref | (jnp.uint32(1) << (13 - t).astype(jnp.uint32))
      cnt = jnp.sum((tie & (lo >= cand)).astype(_i32))
      return jnp.where(cnt >= r, cand, pref)
    lstar = lax.fori_loop(0, 14, lstep, jnp.uint32(0))

    keepn = ((hi > hstar) | (tie & (lo >= lstar))).astype(_f32)
    keepn_ref[...] = keepn
    pn = jnp.sqrt(jnp.sum(p[...] * p[...]))
    scale_ref[...] = jnp.tanh(s / (pn + 1e-16)) * keepn

  return pl.pallas_call(
      body,
      out_shape=[
          jax.ShapeDtypeStruct((80, 128), _f32),
          jax.ShapeDtypeStruct((80, 128), _f32),
      ],
      name=f"topk_{k}",
  )


def _make_pool(kn):
  bm = 256
  nrow = NP // bm

  def body(h, scale, keep, x3_ref, ro_ref):
    i = pl.program_id(0)
    c = pl.program_id(1)
    xn = h[:, pl.ds(c * F, F)] * scale[...]
    x3_ref[...] = xn
    masked = jnp.where(keep[...] > 0.0, xn, -3.4e38)
    cmax = jnp.max(masked, axis=0, keepdims=True)
    csum = jnp.sum(xn, axis=0, keepdims=True)
    cur = jnp.concatenate([cmax, csum], axis=0)[None]

    @pl.when(i == 0)
    def _():
      ro_ref[pl.ds(c, 1)] = cur

    @pl.when(i > 0)
    def _():
      prev = ro_ref[pl.ds(c, 1)]
      mx = jnp.maximum(prev[0, 0:1], cmax)
      sm = prev[0, 1:2] + csum
      ro_ref[pl.ds(c, 1)] = jnp.concatenate([mx, sm], axis=0)[None]

    @pl.when(i == nrow - 1)
    def _():
      prev = ro_ref[pl.ds(c, 1)]
      ro_ref[pl.ds(c, 1)] = jnp.concatenate(
          [prev[0, 0:1], prev[0, 1:2] * (1.0 / kn)], axis=0)[None]

  return pl.pallas_call(
      body,
      grid=(nrow, 8),
      in_specs=[
          pl.BlockSpec((bm, H), lambda i, c: (i, 0)),
          pl.BlockSpec((bm, 1), lambda i, c: (i, 0)),
          pl.BlockSpec((bm, 1), lambda i, c: (i, 0)),
      ],
      out_specs=[
          pl.BlockSpec((bm, F), lambda i, c: (c * nrow + i, 0)),
          pl.BlockSpec((8, 2, F), lambda i, c: (0, 0, 0)),
      ],
      out_shape=[
          jax.ShapeDtypeStruct((8 * NP, F), _f32),
          jax.ShapeDtypeStruct((8, 2, F), _f32),
      ],
      name=f"pool_{kn}",
  )


_edge_agg_1 = _make_edge_agg(1)
_edge_agg_8 = _make_edge_agg(8)
_revalidate = _make_revalidate()
_xr_d = _make_xr(D)
_xr_cm = _make_xr_cm()
_sage_d = _make_sage(D)
_sage_h = _make_sage(H)
_topk = {k: _make_topk(k) for k in (8000, 6400, 5120)}
_pool = {k: _make_pool(k) for k in (8000, 6400, 5120)}


def kernel(x, edge_index, batch, w1_l, w1_r, b1, p1, w2_l, w2_r, b2, p2,
           w3_l, w3_r, b3, p3):
  del batch
  xp = jnp.zeros((NP, D), _f32).at[:N].set(x)
  src = edge_index[0].astype(_i32)
  dst = edge_index[1].astype(_i32)
  npad = EP - E
  pad_dum = N + (jnp.arange(npad, dtype=_i32) % 32)
  srcp = jnp.concatenate([src, pad_dum])
  dstp = jnp.concatenate([dst, jnp.full((npad,), NP - 1, _i32)])
  valid = jnp.concatenate([jnp.ones((E,), _f32), jnp.zeros((npad,), _f32)])
  dst2 = dstp.reshape(EP // 128, 128)
  keep = jnp.concatenate([jnp.ones((N,), _f32), jnp.zeros((NP - N,), _f32)])

  srcef = srcp
  dstcur = dstp
  valtile = valid
  counts = jnp.full((32, 16), EPT, _i32)
  x3 = xp
  result = jnp.zeros((1, 2 * H), _f32)

  layers = [
      (w1_l, w1_r, b1, p1, _sage_d, _edge_agg_1, 8000),
      (w2_l, w2_r, b2, p2, _sage_h, _edge_agg_8, 6400),
      (w3_l, w3_r, b3, p3, _sage_h, _edge_agg_8, 5120),
  ]
  for li, (wl, wr, b, p, sage, eagg, kn) in enumerate(layers):
    if li == 0:
      xr = _xr_d(xp, wr, b.reshape(1, H))
    else:
      xr = _xr_cm(x3, wr, b.reshape(1, H))
    aggp, cntp = eagg(x3, srcef, dstcur, counts)
    h, sraw = sage(aggp, cntp.reshape(2, NP, 1), xr, wl, p.reshape(H, 1))
    keepn2, scale2 = _topk[kn](sraw.reshape(80, 128), keep.reshape(80, 128),
                               p.reshape(8, 128))
    keepn = keepn2.reshape(NP)
    x3, ro = _pool[kn](h, scale2.reshape(NP, 1), keepn.reshape(NP, 1))
    result = result + jnp.concatenate(
        [ro[:, 0].reshape(1, H), ro[:, 1].reshape(1, H)], axis=1)
    if li < 2:
      srcef, dstcur, valtile, valid, counts = _revalidate(
          keepn, srcp, dstp, valid)
      keep = keepn
  return result


# speedup vs baseline: 1.2904x; 1.0002x over previous
import jax
import jax.numpy as jnp
from jax import lax
from jax.experimental import pallas as pl
from jax.experimental.pallas import tpu as pltpu
from jax.experimental.pallas import tpu_sc as plsc

N = 10000
E = 160000
D = 128
H = 1024

NP = 10240
F = 128
EP = 163840
EPT = EP // 32
BB = 128
NSTRIPE = NP // 16

_f32 = jnp.float32
_i32 = jnp.int32


def _make_edge_agg(nch):
  mesh = plsc.VectorSubcoreMesh(core_axis_name="c", subcore_axis_name="s")

  def body(x3, srcef, dstf, counts, aggp, cntp,
           aggsp, cntsp, idxf, idxb, dstst, vones, gbuf, cbuf, ctile, sem):
    cid = lax.axis_index("c")
    sid = lax.axis_index("s")
    tid = cid * 16 + sid
    ebase = pl.multiple_of(cid * (EP // 2) + sid * EPT, EPT)
    r0 = pl.multiple_of(sid * NSTRIPE, NSTRIPE)

    pltpu.sync_copy(srcef.at[pl.ds(ebase, EPT)], idxf)
    pltpu.sync_copy(dstf.at[pl.ds(ebase, EPT)], dstst)
    pltpu.sync_copy(counts.at[tid], ctile)
    nb = (ctile[...][0] + (BB - 1)) // BB

    def ofill(t, carry):
      vones[pl.ds(t * 16, 16)] = jnp.ones((16,), _f32)
      return carry
    lax.fori_loop(0, BB // 16, ofill, 0)

    for ch in range(nch):
      def zfill(t, carry):
        gbuf[0, t // 8, pl.ds((t % 8) * 16, 16)] = jnp.zeros((16,), _f32)
        return carry
      lax.fori_loop(0, 128 * 8, zfill, 0)
      for m in range(NSTRIPE // 128):
        pltpu.sync_copy(gbuf.at[0], aggsp.at[pl.ds(r0 + m * 128, 128)])
      if ch == 0:
        for m in range(NSTRIPE // 128):
          pltpu.sync_copy(gbuf.at[0, 0], cntsp.at[pl.ds(r0 + m * 128, 128)])
      plsc.subcore_barrier()

      coff = ch * NP

      def gsrc(j, par):
        if nch > 1:
          return x3.at[idxb.at[par]]
        return x3.at[idxf.at[pl.ds(j * BB, BB)]]

      def start(j):
        par = j % 2
        if nch > 1:
          def afill(t, carry2):
            idxb[par, pl.ds(t * 16, 16)] = (
                idxf[pl.ds(j * BB + t * 16, 16)] + coff)
            return carry2
          lax.fori_loop(0, BB // 16, afill, 0)
        pltpu.async_copy(gsrc(j, par), gbuf.at[par], sem.at[par])

      @pl.when(nb > 0)
      def _():
        start(0)

      def batch(j, carry):
        @pl.when(j + 1 < nb)
        def _():
          start(j + 1)
        par = j % 2
        pltpu.make_async_copy(gsrc(j, par), gbuf.at[par], sem.at[par]).wait()
        pltpu.sync_copy(gbuf.at[par],
                        aggsp.at[dstst.at[pl.ds(j * BB, BB)]], add=True)
        return carry
      lax.fori_loop(0, nb, batch, 0)

      if ch == 0:
        def cbatch(j, carry):
          pltpu.sync_copy(vones,
                          cntsp.at[dstst.at[pl.ds(j * BB, BB)]], add=True)
          return carry
        lax.fori_loop(0, nb, cbatch, 0)

      plsc.subcore_barrier()

      for m in range(NSTRIPE // 128):
        pltpu.sync_copy(aggsp.at[pl.ds(r0 + m * 128, 128)], gbuf.at[0])
        pltpu.sync_copy(
            gbuf.at[0],
            aggp.at[cid, pl.ds(r0 + m * 128, 128), pl.ds(ch * F, F)])
      if ch == 0:
        pltpu.sync_copy(cntsp.at[pl.ds(r0, NSTRIPE)], cbuf)
        pltpu.sync_copy(cbuf, cntp.at[cid, pl.ds(r0, NSTRIPE)])

  return pl.kernel(
      body,
      out_type=[
          jax.ShapeDtypeStruct((2, NP, nch * F), _f32),
          jax.ShapeDtypeStruct((2, NP), _f32),
      ],
      mesh=mesh,
      scratch_types=[
          pltpu.VMEM_SHARED((NP, F), _f32),
          pltpu.VMEM_SHARED((NP,), _f32),
          pltpu.VMEM((EPT,), _i32),
          pltpu.VMEM((2, BB), _i32),
          pltpu.VMEM((EPT,), _i32),
          pltpu.VMEM((BB,), _f32),
          pltpu.VMEM((2, BB, F), _f32),
          pltpu.VMEM((NSTRIPE,), _f32),
          pltpu.VMEM((16,), _i32),
          pltpu.SemaphoreType.DMA((2,)),
      ],
      name=f"edge_agg_{nch}",
  )


def _make_revalidate():
  mesh = plsc.VectorSubcoreMesh(core_axis_name="c", subcore_axis_name="s")

  def body(keep, srcp, dstp, valp, srcc_o, dstc_o, valc_o, valn_o, cnts_o,
           keepst, sst, dstt, vst_, sout, dout, vout, vfull, cb16):
    cid = lax.axis_index("c")
    sid = lax.axis_index("s")
    tid = cid * 16 + sid
    ebase = pl.multiple_of(cid * (EP // 2) + sid * EPT, EPT)

    pltpu.sync_copy(keep, keepst)
    pltpu.sync_copy(srcp.at[pl.ds(ebase, EPT)], sst)
    pltpu.sync_copy(dstp.at[pl.ds(ebase, EPT)], dstt)
    pltpu.sync_copy(valp.at[pl.ds(ebase, EPT)], vst_)

    iota16 = lax.iota(_i32, 16)
    zeros16i = jnp.zeros((16,), _i32)
    ones16 = jnp.ones((16,), _f32)

    def pre(t, c):
      sout[pl.ds(t * 16, 16)] = N + ((iota16 + t) % 32)
      dout[pl.ds(t * 16, 16)] = zeros16i + (NP - 1)
      vout[pl.ds(t * 16, 16)] = jnp.zeros((16,), _f32)
      return c
    lax.fori_loop(0, EPT // 16, pre, 0)

    def step(t, cnt):
      s16 = sst[pl.ds(t * 16, 16)]
      d16 = dstt[pl.ds(t * 16, 16)]
      v16 = vst_[pl.ds(t * 16, 16)]
      ks = plsc.load_gather(keepst, [s16])
      kd = plsc.load_gather(keepst, [d16])
      v = v16 * ks * kd
      vfull[pl.ds(t * 16, 16)] = v
      m = v > 0.0
      vi = jnp.where(m, 1, 0)
      pos = plsc.cumsum(vi) + (cnt - 1)
      plsc.store_scatter(sout, [pos], s16, mask=m)
      plsc.store_scatter(dout, [pos], d16, mask=m)
      plsc.store_scatter(vout, [pos], ones16, mask=m)
      return cnt + jnp.sum(vi)
    cnt = lax.fori_loop(0, EPT // 16, step, 0)

    cb16[...] = zeros16i + cnt
    pltpu.sync_copy(sout, srcc_o.at[pl.ds(ebase, EPT)])
    pltpu.sync_copy(dout, dstc_o.at[pl.ds(ebase, EPT)])
    pltpu.sync_copy(vout, valc_o.at[pl.ds(ebase, EPT)])
    pltpu.sync_copy(vfull, valn_o.at[pl.ds(ebase, EPT)])
    pltpu.sync_copy(cb16, cnts_o.at[tid])

  return pl.kernel(
      body,
      out_type=[
          jax.ShapeDtypeStruct((EP,), _i32),
          jax.ShapeDtypeStruct((EP,), _i32),
          jax.ShapeDtypeStruct((EP,), _f32),
          jax.ShapeDtypeStruct((EP,), _f32),
          jax.ShapeDtypeStruct((32, 16), _i32),
      ],
      mesh=mesh,
      scratch_types=[
          pltpu.VMEM((NP,), _f32),
          pltpu.VMEM((EPT,), _i32),
          pltpu.VMEM((EPT,), _i32),
          pltpu.VMEM((EPT,), _f32),
          pltpu.VMEM((EPT,), _i32),
          pltpu.VMEM((EPT,), _i32),
          pltpu.VMEM((EPT,), _f32),
          pltpu.VMEM((EPT,), _f32),
          pltpu.VMEM((16,), _i32),
      ],
      compiler_params=pltpu.CompilerParams(needs_layout_passes=False),
      name="revalidate",
  )


def _make_xr(din):
  bm = 256

  def body(x, wr, b, xr_ref):
    xr_ref[...] = (jnp.dot(x[...], wr[...], preferred_element_type=_f32)
                   + b[...])

  return pl.pallas_call(
      body,
      grid=(NP // bm,),
      in_specs=[
          pl.BlockSpec((bm, din), lambda i: (i, 0)),
          pl.BlockSpec((din, H), lambda i: (0, 0)),
          pl.BlockSpec((1, H), lambda i: (0, 0)),
      ],
      out_specs=pl.BlockSpec((bm, H), lambda i: (i, 0)),
      out_shape=jax.ShapeDtypeStruct((NP, H), _f32),
      name=f"xr_{din}",
  )


def _make_xr_cm():
  bm = 256
  nrow = NP // bm

  def body(x3, wr, b, xr_ref):
    c = pl.program_id(1)

    @pl.when(c == 0)
    def _():
      xr_ref[...] = jnp.zeros((bm, H), _f32) + b[...]

    xr_ref[...] += jnp.dot(x3[...], wr[...], preferred_element_type=_f32)

  return pl.pallas_call(
      body,
      grid=(nrow, 8),
      in_specs=[
          pl.BlockSpec((bm, F), lambda i, c: (c * nrow + i, 0)),
          pl.BlockSpec((F, H), lambda i, c: (c, 0)),
          pl.BlockSpec((1, H), lambda i, c: (0, 0)),
      ],
      out_specs=pl.BlockSpec((bm, H), lambda i, c: (i, 0)),
      out_shape=jax.ShapeDtypeStruct((NP, H), _f32),
      name="xr_cm",
  )


def _make_sage(din):
  bm = 256

  def body(aggp, cntp, xr, wl, p, h_ref, sraw_ref):
    agg = aggp[0] + aggp[1]
    c = cntp[0] + cntp[1]
    mean = jnp.where(c > 0.0, agg / jnp.maximum(c, 1.0), 0.0)
    hm = jnp.dot(mean, wl[...], preferred_element_type=_f32) + xr[...]
    h = jnp.maximum(hm, 0.0)
    h_ref[...] = h
    sraw_ref[...] = jnp.dot(h, p[...], preferred_element_type=_f32)

  return pl.pallas_call(
      body,
      grid=(NP // bm,),
      in_specs=[
          pl.BlockSpec((2, bm, din), lambda i: (0, i, 0)),
          pl.BlockSpec((2, bm, 1), lambda i: (0, i, 0)),
          pl.BlockSpec((bm, H), lambda i: (i, 0)),
          pl.BlockSpec((din, H), lambda i: (0, 0)),
          pl.BlockSpec((H, 1), lambda i: (0, 0)),
      ],
      out_specs=[
          pl.BlockSpec((bm, H), lambda i: (i, 0)),
          pl.BlockSpec((bm, 1), lambda i: (i, 0)),
      ],
      out_shape=[
          jax.ShapeDtypeStruct((NP, H), _f32),
          jax.ShapeDtypeStruct((NP, 1), _f32),
      ],
      name=f"sage_{din}",
  )


def _make_topk(k):
  def body(sraw, keep, p, keepn_ref, scale_ref):
    s = sraw[...]
    kp = keep[...]
    bits = lax.bitcast_convert_type(s, jnp.uint32)
    top = jnp.uint32(0x80000000)
    sortable = jnp.where((bits & top) != 0, ~bits, bits | top)
    hi = jnp.where(kp > 0.0, sortable, jnp.uint32(0))
    ridx = (lax.broadcasted_iota(_i32, (80, 128), 0) * 128
            + lax.broadcasted_iota(_i32, (80, 128), 1))
    lo = (NP - ridx).astype(jnp.uint32)

    def hstep(t, pref):
      cand = pref | (jnp.uint32(1) << (31 - t).astype(jnp.uint32))
      cnt = jnp.sum((hi >= cand).astype(_i32))
      return jnp.where(cnt >= k, cand, pref)
    hstar = lax.fori_loop(0, 32, hstep, jnp.uint32(0))

    ngt = jnp.sum((hi > hstar).astype(_i32))
    r = k - ngt
    tie = hi == hstar

    def lstep(t, pref):
      cand = pref | (jnp.uint32(1) << (13 - t).astype(jnp.uint32))
      cnt = jnp.sum((tie & (lo >= cand)).astype(_i32))
      return jnp.where(cnt >= r, cand, pref)
    lstar = lax.fori_loop(0, 14, lstep, jnp.uint32(0))

    keepn = ((hi > hstar) | (tie & (lo >= lstar))).astype(_f32)
    keepn_ref[...] = keepn
    pn = jnp.sqrt(jnp.sum(p[...] * p[...]))
    scale_ref[...] = jnp.tanh(s / (pn + 1e-16)) * keepn

  return pl.pallas_call(
      body,
      out_shape=[
          jax.ShapeDtypeStruct((80, 128), _f32),
          jax.ShapeDtypeStruct((80, 128), _f32),
      ],
      name=f"topk_{k}",
  )


def _make_pool(kn):
  bm = 256
  nrow = NP // bm

  def body(h, scale, keep, x3_ref, ro_ref):
    i = pl.program_id(0)
    c = pl.program_id(1)
    xn = h[:, pl.ds(c * F, F)] * scale[...]
    x3_ref[...] = xn
    masked = jnp.where(keep[...] > 0.0, xn, -3.4e38)
    cmax = jnp.max(masked, axis=0, keepdims=True)
    csum = jnp.sum(xn, axis=0, keepdims=True)
    cur = jnp.concatenate([cmax, csum], axis=0)[None]

    @pl.when(i == 0)
    def _():
      ro_ref[pl.ds(c, 1)] = cur

    @pl.when(i > 0)
    def _():
      prev = ro_ref[pl.ds(c, 1)]
      mx = jnp.maximum(prev[0, 0:1], cmax)
      sm = prev[0, 1:2] + csum
      ro_ref[pl.ds(c, 1)] = jnp.concatenate([mx, sm], axis=0)[None]

    @pl.when(i == nrow - 1)
    def _():
      prev = ro_ref[pl.ds(c, 1)]
      ro_ref[pl.ds(c, 1)] = jnp.concatenate(
          [prev[0, 0:1], prev[0, 1:2] * (1.0 / kn)], axis=0)[None]

  return pl.pallas_call(
      body,
      grid=(nrow, 8),
      in_specs=[
          pl.BlockSpec((bm, H), lambda i, c: (i, 0)),
          pl.BlockSpec((bm, 1), lambda i, c: (i, 0)),
          pl.BlockSpec((bm, 1), lambda i, c: (i, 0)),
      ],
      out_specs=[
          pl.BlockSpec((bm, F), lambda i, c: (c * nrow + i, 0)),
          pl.BlockSpec((8, 2, F), lambda i, c: (0, 0, 0)),
      ],
      out_shape=[
          jax.ShapeDtypeStruct((8 * NP, F), _f32),
          jax.ShapeDtypeStruct((8, 2, F), _f32),
      ],
      name=f"pool_{kn}",
  )


_edge_agg_1 = _make_edge_agg(1)
_edge_agg_8 = _make_edge_agg(8)
_revalidate = _make_revalidate()
_xr_d = _make_xr(D)
_xr_cm = _make_xr_cm()
_sage_d = _make_sage(D)
_sage_h = _make_sage(H)
_topk = {k: _make_topk(k) for k in (8000, 6400, 5120)}
_pool = {k: _make_pool(k) for k in (8000, 6400, 5120)}


def kernel(x, edge_index, batch, w1_l, w1_r, b1, p1, w2_l, w2_r, b2, p2,
           w3_l, w3_r, b3, p3):
  del batch
  xp = jnp.zeros((NP, D), _f32).at[:N].set(x)
  src = edge_index[0].astype(_i32)
  dst = edge_index[1].astype(_i32)
  npad = EP - E
  pad_dum = N + (jnp.arange(npad, dtype=_i32) % 32)
  srcp = jnp.concatenate([src, pad_dum])
  dstp = jnp.concatenate([dst, jnp.full((npad,), NP - 1, _i32)])
  valid = jnp.concatenate([jnp.ones((E,), _f32), jnp.zeros((npad,), _f32)])
  keep = jnp.concatenate([jnp.ones((N,), _f32), jnp.zeros((NP - N,), _f32)])

  srcef = srcp
  dstcur = dstp
  counts = jnp.full((32, 16), EPT, _i32)
  x3 = xp
  result = jnp.zeros((1, 2 * H), _f32)

  layers = [
      (w1_l, w1_r, b1, p1, _sage_d, _edge_agg_1, 8000),
      (w2_l, w2_r, b2, p2, _sage_h, _edge_agg_8, 6400),
      (w3_l, w3_r, b3, p3, _sage_h, _edge_agg_8, 5120),
  ]
  for li, (wl, wr, b, p, sage, eagg, kn) in enumerate(layers):
    if li == 0:
      xr = _xr_d(xp, wr, b.reshape(1, H))
    else:
      xr = _xr_cm(x3, wr, b.reshape(1, H))
    aggp, cntp = eagg(x3, srcef, dstcur, counts)
    h, sraw = sage(aggp, cntp.reshape(2, NP, 1), xr, wl, p.reshape(H, 1))
    keepn2, scale2 = _topk[kn](sraw.reshape(80, 128), keep.reshape(80, 128),
                               p.reshape(8, 128))
    keepn = keepn2.reshape(NP)
    x3, ro = _pool[kn](h, scale2.reshape(NP, 1), keepn.reshape(NP, 1))
    result = result + jnp.concatenate(
        [ro[:, 0].reshape(1, H), ro[:, 1].reshape(1, H)], axis=1)
    if li < 2:
      srcef, dstcur, _valc, valid, counts = _revalidate(
          keepn, srcp, dstp, valid)
      keep = keepn
  return result
